# baseline (speedup 1.0000x reference)
import functools

import jax
import jax.numpy as jnp
from jax.experimental import pallas as pl
from jax.experimental.pallas import tpu as pltpu

N = 20000
E = 50000
F_IN = 30
D_E = 16
N_GRAPHS = 512

T_E = 512
EPAD = 53248
BN = 2048
NPAD = 20480


def _relu(v):
    return jnp.maximum(v, 0.0)


def _edge_body(e_ref, xg_ref, w1_ref, b1_ref, w2_ref, b2_ref, m_ref, o_ref,
               *, c):
    h = _relu(jnp.dot(e_ref[...], w1_ref[...],
                      preferred_element_type=jnp.float32) + b1_ref[...])
    h = _relu(jnp.dot(h, w2_ref[...],
                      preferred_element_type=jnp.float32) + b2_ref[...])
    lane = jax.lax.broadcasted_iota(jnp.int32, h.shape, 1)
    h = jnp.where(lane == 30, 1.0, h)
    p = (xg_ref[...][:, :, None] * h[:, None, :]).reshape(T_E, 32 * 32)
    msg = jnp.dot(p, m_ref[...], preferred_element_type=jnp.float32)
    row = jax.lax.broadcasted_iota(jnp.int32, msg.shape, 0) + pl.program_id(0) * T_E
    o_ref[...] = jnp.where(row < E, msg, 0.0)


def _edge_call(e_pad, xg, w1p, b1p, w2p, b2p, mp, c):
    grid = EPAD // T_E
    return pl.pallas_call(
        functools.partial(_edge_body, c=c),
        grid=(grid,),
        in_specs=[
            pl.BlockSpec((T_E, D_E), lambda n: (n, 0)),
            pl.BlockSpec((T_E, 32), lambda n: (n, 0)),
            pl.BlockSpec((D_E, 32), lambda n: (0, 0)),
            pl.BlockSpec((1, 32), lambda n: (0, 0)),
            pl.BlockSpec((32, 32), lambda n: (0, 0)),
            pl.BlockSpec((1, 32), lambda n: (0, 0)),
            pl.BlockSpec((32 * 32, c), lambda n: (0, 0)),
        ],
        out_specs=pl.BlockSpec((T_E, c), lambda n: (n, 0)),
        out_shape=jax.ShapeDtypeStruct((EPAD, c), jnp.float32),
    )(e_pad, xg, w1p, b1p, w2p, b2p, mp)


def _node1_body(agg_ref, x_ref, root_ref, bias_ref, o_ref):
    o_ref[...] = _relu(
        agg_ref[...]
        + jnp.dot(x_ref[...], root_ref[...], preferred_element_type=jnp.float32)
        + bias_ref[...])


def _node1_call(agg, x, root, bias):
    grid = NPAD // BN
    return pl.pallas_call(
        _node1_body,
        grid=(grid,),
        in_specs=[
            pl.BlockSpec((BN, 32), lambda n: (n, 0)),
            pl.BlockSpec((BN, F_IN), lambda n: (n, 0)),
            pl.BlockSpec((F_IN, 32), lambda n: (0, 0)),
            pl.BlockSpec((1, 32), lambda n: (0, 0)),
        ],
        out_specs=pl.BlockSpec((BN, 32), lambda n: (n, 0)),
        out_shape=jax.ShapeDtypeStruct((NPAD, 32), jnp.float32),
    )(agg, x, root, bias)


def _node2_body(agg_ref, h1_ref, root_ref, bias_ref, gw_ref, o_ref):
    h2 = _relu(
        agg_ref[...]
        + jnp.dot(h1_ref[...], root_ref[...], preferred_element_type=jnp.float32)
        + bias_ref[...])
    o_ref[...] = jnp.dot(h2, gw_ref[...], preferred_element_type=jnp.float32)


def _node2_call(agg, h1, root, bias, gw):
    grid = NPAD // BN
    return pl.pallas_call(
        _node2_body,
        grid=(grid,),
        in_specs=[
            pl.BlockSpec((BN, 64), lambda n: (n, 0)),
            pl.BlockSpec((BN, 32), lambda n: (n, 0)),
            pl.BlockSpec((32, 64), lambda n: (0, 0)),
            pl.BlockSpec((1, 64), lambda n: (0, 0)),
            pl.BlockSpec((64, 32), lambda n: (0, 0)),
        ],
        out_specs=pl.BlockSpec((BN, 32), lambda n: (n, 0)),
        out_shape=jax.ShapeDtypeStruct((NPAD, 32), jnp.float32),
    )(agg, h1, root, bias, gw)


def _pool_body(agg_ref, i_ref, b_ref, o_ref):
    n = pl.program_id(0)
    h3 = _relu(agg_ref[...] + b_ref[...])
    iv = i_ref[0]
    onehot = jnp.where(
        jax.lax.broadcasted_iota(jnp.int32, (N_GRAPHS, BN), 0) == iv, 1.0, 0.0)
    contrib = jnp.dot(onehot, h3, preferred_element_type=jnp.float32)

    @pl.when(n == 0)
    def _():
        o_ref[...] = jnp.zeros_like(o_ref)

    o_ref[...] += contrib


def _pool_call(agg3, i3, gcn_b):
    grid = NPAD // BN
    return pl.pallas_call(
        _pool_body,
        grid=(grid,),
        in_specs=[
            pl.BlockSpec((BN, 32), lambda n: (n, 0)),
            pl.BlockSpec((1, 1, BN), lambda n: (n, 0, 0)),
            pl.BlockSpec((1, 32), lambda n: (0, 0)),
        ],
        out_specs=pl.BlockSpec((N_GRAPHS, 32), lambda n: (0, 0)),
        out_shape=jax.ShapeDtypeStruct((N_GRAPHS, 32), jnp.float32),
    )(agg3, i3, gcn_b)


def _make_m(wk, bk, f, c):
    wk_aug = jnp.concatenate(
        [wk, bk[None, :], jnp.zeros((1, f * c), jnp.float32)], axis=0)
    a = wk_aug.reshape(32, f, c)
    if f < 32:
        a = jnp.pad(a, ((0, 0), (0, 32 - f), (0, 0)))
    a = a.transpose(1, 0, 2)
    return a.reshape(32 * 32, c)


def _pad2(w, rows, cols):
    return jnp.pad(w, ((0, rows - w.shape[0]), (0, cols - w.shape[1])))


def kernel(x, edge_index, e, i, m1_W1, m1_b1, m1_W2, m1_b2, m1_Wk, m1_bk,
           m1_root, m1_bias, m2_W1, m2_b1, m2_W2, m2_b2, m2_Wk, m2_bk,
           m2_root, m2_bias, gcn_W, gcn_b):
    src = edge_index[0]
    dst = edge_index[1]
    src_pad = jnp.concatenate([src, jnp.zeros((EPAD - E,), jnp.int32)])

    e_pad = jnp.pad(e, ((0, EPAD - E), (0, 0)))
    x32 = jnp.pad(x, ((0, 0), (0, 32 - F_IN)))

    m1w1 = _pad2(m1_W1, D_E, 32)
    m1b1 = jnp.pad(m1_b1, (0, 2)).reshape(1, 32)
    m1w2 = _pad2(m1_W2, 32, 32)
    m1b2 = jnp.pad(m1_b2, (0, 2)).reshape(1, 32)
    m1m = _make_m(m1_Wk, m1_bk, F_IN, 32)
    m2w1 = _pad2(m2_W1, D_E, 32)
    m2b1 = jnp.pad(m2_b1, (0, 2)).reshape(1, 32)
    m2w2 = _pad2(m2_W2, 32, 32)
    m2b2 = jnp.pad(m2_b2, (0, 2)).reshape(1, 32)
    m2m = _make_m(m2_Wk, m2_bk, 32, 64)

    xg = jnp.take(x32, src_pad, axis=0)
    msg1 = _edge_call(e_pad, xg, m1w1, m1b1, m1w2, m1b2, m1m, 32)
    agg1 = jnp.zeros((NPAD, 32), jnp.float32).at[dst].add(msg1[:E])
    x_n = jnp.pad(x, ((0, NPAD - N), (0, 0)))
    h1 = _node1_call(agg1, x_n, m1_root, m1_bias.reshape(1, 32))

    h1g = jnp.take(h1, src_pad, axis=0)
    msg2 = _edge_call(e_pad, h1g, m2w1, m2b1, m2w2, m2b2, m2m, 64)
    agg2 = jnp.zeros((NPAD, 64), jnp.float32).at[dst].add(msg2[:E])
    hw = _node2_call(agg2, h1, m2_root, m2_bias.reshape(1, 64), gcn_W)

    hwg = jnp.take(hw, src, axis=0)
    agg3 = jnp.zeros((NPAD, 32), jnp.float32).at[dst].add(hwg)
    i_pad = jnp.concatenate(
        [i, jnp.full((NPAD - N,), N_GRAPHS, jnp.int32)]).reshape(NPAD // BN, 1, BN)
    return _pool_call(agg3, i_pad, gcn_b.reshape(1, 32))


# speedup vs baseline: 1.7228x; 1.7228x over previous
import functools

import jax
import jax.numpy as jnp
from jax import lax
from jax.experimental import pallas as pl
from jax.experimental.pallas import tpu as pltpu
from jax.experimental.pallas import tpu_sc as plsc

N = 20000
E = 50000
F_IN = 30
D_E = 16
N_GRAPHS = 512

T_E = 512
EPAD = 53248
BN = 2048
NPAD = 20480

NC = 2
NS = 16
NW = NC * NS
CHUNK = 128
CH = EPAD // (NW * CHUNK)
PERW = CH * CHUNK
HALF = NPAD // 2
IMG = HALF + 128
SSTRIPE = IMG // NS
DSTRIPE = HALF // NS


def _relu(v):
    return jnp.maximum(v, 0.0)


def _sc_mesh():
    return plsc.VectorSubcoreMesh(core_axis_name="c", subcore_axis_name="s")


def _sc_gather(table, idx2):
    @functools.partial(
        pl.kernel,
        out_type=jax.ShapeDtypeStruct((EPAD, 32), jnp.float32),
        mesh=_sc_mesh(),
        compiler_params=pltpu.CompilerParams(use_tc_tiling_on_sc=False),
        scratch_types=[
            pltpu.VMEM((CH, CHUNK), jnp.int32),
            pltpu.VMEM((PERW, 32), jnp.float32),
            pltpu.SemaphoreType.DMA,
        ],
    )
    def k(table_hbm, idx_hbm, out_hbm, idx_v, rows_v, sem):
        wid = lax.axis_index("s") * NC + lax.axis_index("c")
        pltpu.sync_copy(idx_hbm.at[wid], idx_v)
        cps = [
            pltpu.async_copy(table_hbm.at[idx_v.at[j]],
                             rows_v.at[pl.ds(j * CHUNK, CHUNK)], sem)
            for j in range(CH)
        ]
        for d in cps:
            d.wait()
        pltpu.sync_copy(rows_v, out_hbm.at[pl.ds(wid * PERW, PERW)])

    return k(table, idx2)


def _sc_scatter(msg, dsts, zrows, c):
    @functools.partial(
        pl.kernel,
        out_type=jax.ShapeDtypeStruct((NPAD, c), jnp.float32),
        mesh=_sc_mesh(),
        compiler_params=pltpu.CompilerParams(use_tc_tiling_on_sc=False),
        scratch_types=[
            pltpu.VMEM((CH, CHUNK), jnp.int32),
            pltpu.VMEM((PERW, c), jnp.float32),
            pltpu.VMEM_SHARED((IMG, c), jnp.float32),
        ],
    )
    def k(msg_hbm, dst_hbm, z_hbm, out_hbm, dst_v, msg_v, shared):
        cc = lax.axis_index("c")
        s = lax.axis_index("s")
        pltpu.sync_copy(z_hbm.at[pl.ds(s * SSTRIPE, SSTRIPE)],
                        shared.at[pl.ds(s * SSTRIPE, SSTRIPE)])
        plsc.subcore_barrier()
        for half in range(2):
            w = s * 2 + half
            pltpu.sync_copy(dst_hbm.at[cc].at[w], dst_v)
            pltpu.sync_copy(msg_hbm.at[pl.ds(w * PERW, PERW)], msg_v)
            for j in range(CH):
                pltpu.sync_copy(msg_v.at[pl.ds(j * CHUNK, CHUNK)],
                                shared.at[dst_v.at[j]], add=True)
        plsc.subcore_barrier()
        pltpu.sync_copy(shared.at[pl.ds(s * DSTRIPE, DSTRIPE)],
                        out_hbm.at[pl.ds(cc * HALF + s * DSTRIPE, DSTRIPE)])

    return k(msg, dsts, zrows)


def _sc_gather_scatter(hw, src2, dsts, zrows):
    @functools.partial(
        pl.kernel,
        out_type=jax.ShapeDtypeStruct((NPAD, 32), jnp.float32),
        mesh=_sc_mesh(),
        compiler_params=pltpu.CompilerParams(use_tc_tiling_on_sc=False),
        scratch_types=[
            pltpu.VMEM((CH, CHUNK), jnp.int32),
            pltpu.VMEM((CH, CHUNK), jnp.int32),
            pltpu.VMEM((PERW, 32), jnp.float32),
            pltpu.VMEM_SHARED((IMG, 32), jnp.float32),
            pltpu.SemaphoreType.DMA,
        ],
    )
    def k(hw_hbm, src_hbm, dst_hbm, z_hbm, out_hbm, src_v, dst_v, rows_v,
          shared, sem):
        cc = lax.axis_index("c")
        s = lax.axis_index("s")
        pltpu.sync_copy(z_hbm.at[pl.ds(s * SSTRIPE, SSTRIPE)],
                        shared.at[pl.ds(s * SSTRIPE, SSTRIPE)])
        plsc.subcore_barrier()
        for half in range(2):
            w = s * 2 + half
            pltpu.sync_copy(src_hbm.at[w], src_v)
            pltpu.sync_copy(dst_hbm.at[cc].at[w], dst_v)
            cps = [
                pltpu.async_copy(hw_hbm.at[src_v.at[j]],
                                 rows_v.at[pl.ds(j * CHUNK, CHUNK)], sem)
                for j in range(CH)
            ]
            for d in cps:
                d.wait()
            for j in range(CH):
                pltpu.sync_copy(rows_v.at[pl.ds(j * CHUNK, CHUNK)],
                                shared.at[dst_v.at[j]], add=True)
        plsc.subcore_barrier()
        pltpu.sync_copy(shared.at[pl.ds(s * DSTRIPE, DSTRIPE)],
                        out_hbm.at[pl.ds(cc * HALF + s * DSTRIPE, DSTRIPE)])

    return k(hw, src2, dsts, zrows)


def _edge_body(e_ref, xg_ref, w1_ref, b1_ref, w2_ref, b2_ref, m_ref, *o_refs,
               c):
    h = _relu(jnp.dot(e_ref[...], w1_ref[...],
                      preferred_element_type=jnp.float32) + b1_ref[...])
    h = _relu(jnp.dot(h, w2_ref[...],
                      preferred_element_type=jnp.float32) + b2_ref[...])
    lane = jax.lax.broadcasted_iota(jnp.int32, h.shape, 1)
    h = jnp.where(lane == 30, 1.0, h)
    p = (xg_ref[...][:, :, None] * h[:, None, :]).reshape(T_E, 32 * 32)
    msg = jnp.dot(p, m_ref[...], preferred_element_type=jnp.float32)
    row = jax.lax.broadcasted_iota(jnp.int32, msg.shape, 0) + pl.program_id(0) * T_E
    msg = jnp.where(row < E, msg, 0.0)
    for g, o_ref in enumerate(o_refs):
        o_ref[...] = msg[:, g * 32:(g + 1) * 32]


def _edge_call(e_pad, xg, w1p, b1p, w2p, b2p, mp, c):
    grid = EPAD // T_E
    return pl.pallas_call(
        functools.partial(_edge_body, c=c),
        grid=(grid,),
        in_specs=[
            pl.BlockSpec((T_E, D_E), lambda n: (n, 0)),
            pl.BlockSpec((T_E, 32), lambda n: (n, 0)),
            pl.BlockSpec((D_E, 32), lambda n: (0, 0)),
            pl.BlockSpec((1, 32), lambda n: (0, 0)),
            pl.BlockSpec((32, 32), lambda n: (0, 0)),
            pl.BlockSpec((1, 32), lambda n: (0, 0)),
            pl.BlockSpec((32 * 32, c), lambda n: (0, 0)),
        ],
        out_specs=[pl.BlockSpec((T_E, 32), lambda n: (n, 0))] * (c // 32),
        out_shape=[jax.ShapeDtypeStruct((EPAD, 32), jnp.float32)] * (c // 32),
    )(e_pad, xg, w1p, b1p, w2p, b2p, mp)


def _node1_body(agg_ref, x_ref, root_ref, bias_ref, o_ref):
    o_ref[...] = _relu(
        agg_ref[...]
        + jnp.dot(x_ref[...], root_ref[...], preferred_element_type=jnp.float32)
        + bias_ref[...])


def _node1_call(agg, x, root, bias):
    grid = NPAD // BN
    return pl.pallas_call(
        _node1_body,
        grid=(grid,),
        in_specs=[
            pl.BlockSpec((BN, 32), lambda n: (n, 0)),
            pl.BlockSpec((BN, F_IN), lambda n: (n, 0)),
            pl.BlockSpec((F_IN, 32), lambda n: (0, 0)),
            pl.BlockSpec((1, 32), lambda n: (0, 0)),
        ],
        out_specs=pl.BlockSpec((BN, 32), lambda n: (n, 0)),
        out_shape=jax.ShapeDtypeStruct((NPAD, 32), jnp.float32),
    )(agg, x, root, bias)


def _node2_body(agga_ref, aggb_ref, h1_ref, root_ref, bias_ref, gw_ref, o_ref):
    rt = jnp.dot(h1_ref[...], root_ref[...], preferred_element_type=jnp.float32)
    h2a = _relu(agga_ref[...] + rt[:, :32] + bias_ref[...][:, :32])
    h2b = _relu(aggb_ref[...] + rt[:, 32:] + bias_ref[...][:, 32:])
    o_ref[...] = (
        jnp.dot(h2a, gw_ref[...][:32], preferred_element_type=jnp.float32)
        + jnp.dot(h2b, gw_ref[...][32:], preferred_element_type=jnp.float32))


def _node2_call(agga, aggb, h1, root, bias, gw):
    grid = NPAD // BN
    return pl.pallas_call(
        _node2_body,
        grid=(grid,),
        in_specs=[
            pl.BlockSpec((BN, 32), lambda n: (n, 0)),
            pl.BlockSpec((BN, 32), lambda n: (n, 0)),
            pl.BlockSpec((BN, 32), lambda n: (n, 0)),
            pl.BlockSpec((32, 64), lambda n: (0, 0)),
            pl.BlockSpec((1, 64), lambda n: (0, 0)),
            pl.BlockSpec((64, 32), lambda n: (0, 0)),
        ],
        out_specs=pl.BlockSpec((BN, 32), lambda n: (n, 0)),
        out_shape=jax.ShapeDtypeStruct((NPAD, 32), jnp.float32),
    )(agga, aggb, h1, root, bias, gw)


def _pool_body(agg_ref, i_ref, b_ref, o_ref):
    n = pl.program_id(0)
    h3 = _relu(agg_ref[...] + b_ref[...])
    iv = i_ref[0]
    onehot = jnp.where(
        jax.lax.broadcasted_iota(jnp.int32, (N_GRAPHS, BN), 0) == iv, 1.0, 0.0)
    contrib = jnp.dot(onehot, h3, preferred_element_type=jnp.float32)

    @pl.when(n == 0)
    def _():
        o_ref[...] = jnp.zeros_like(o_ref)

    o_ref[...] += contrib


def _pool_call(agg3, i3, gcn_b):
    grid = NPAD // BN
    return pl.pallas_call(
        _pool_body,
        grid=(grid,),
        in_specs=[
            pl.BlockSpec((BN, 32), lambda n: (n, 0)),
            pl.BlockSpec((1, 1, BN), lambda n: (n, 0, 0)),
            pl.BlockSpec((1, 32), lambda n: (0, 0)),
        ],
        out_specs=pl.BlockSpec((N_GRAPHS, 32), lambda n: (0, 0)),
        out_shape=jax.ShapeDtypeStruct((N_GRAPHS, 32), jnp.float32),
    )(agg3, i3, gcn_b)


def _make_m(wk, bk, f, c):
    wk_aug = jnp.concatenate(
        [wk, bk[None, :], jnp.zeros((1, f * c), jnp.float32)], axis=0)
    a = wk_aug.reshape(32, f, c)
    if f < 32:
        a = jnp.pad(a, ((0, 0), (0, 32 - f), (0, 0)))
    a = a.transpose(1, 0, 2)
    return a.reshape(32 * 32, c)


def _pad2(w, rows, cols):
    return jnp.pad(w, ((0, rows - w.shape[0]), (0, cols - w.shape[1])))


def kernel(x, edge_index, e, i, m1_W1, m1_b1, m1_W2, m1_b2, m1_Wk, m1_bk,
           m1_root, m1_bias, m2_W1, m2_b1, m2_W2, m2_b2, m2_Wk, m2_bk,
           m2_root, m2_bias, gcn_W, gcn_b):
    src = edge_index[0]
    dst = edge_index[1]
    src2 = jnp.concatenate(
        [src, jnp.zeros((EPAD - E,), jnp.int32)]).reshape(NW, CH, CHUNK)
    dpad = jnp.concatenate([dst, jnp.full((EPAD - E,), NPAD - 1, jnp.int32)])
    d0 = jnp.where(dpad < HALF, dpad, HALF)
    d1r = dpad - HALF
    d1 = jnp.where((d1r >= 0) & (d1r < HALF), d1r, HALF)
    dsts = jnp.stack([d0, d1]).reshape(NC, NW, CH, CHUNK)
    z32 = jnp.zeros((IMG, 32), jnp.float32)

    e_pad = jnp.pad(e, ((0, EPAD - E), (0, 0)))
    x32 = jnp.pad(x, ((0, 0), (0, 32 - F_IN)))

    m1w1 = _pad2(m1_W1, D_E, 32)
    m1b1 = jnp.pad(m1_b1, (0, 2)).reshape(1, 32)
    m1w2 = _pad2(m1_W2, 32, 32)
    m1b2 = jnp.pad(m1_b2, (0, 2)).reshape(1, 32)
    m1m = _make_m(m1_Wk, m1_bk, F_IN, 32)
    m2w1 = _pad2(m2_W1, D_E, 32)
    m2b1 = jnp.pad(m2_b1, (0, 2)).reshape(1, 32)
    m2w2 = _pad2(m2_W2, 32, 32)
    m2b2 = jnp.pad(m2_b2, (0, 2)).reshape(1, 32)
    m2m = _make_m(m2_Wk, m2_bk, 32, 64)

    xg = _sc_gather(x32, src2)
    (msg1,) = _edge_call(e_pad, xg, m1w1, m1b1, m1w2, m1b2, m1m, 32)
    agg1 = _sc_scatter(msg1, dsts, z32, 32)
    x_n = jnp.pad(x, ((0, NPAD - N), (0, 0)))
    h1 = _node1_call(agg1, x_n, m1_root, m1_bias.reshape(1, 32))

    h1g = _sc_gather(h1, src2)
    msg2a, msg2b = _edge_call(e_pad, h1g, m2w1, m2b1, m2w2, m2b2, m2m, 64)
    agg2a = _sc_scatter(msg2a, dsts, z32, 32)
    agg2b = _sc_scatter(msg2b, dsts, z32, 32)
    hw = _node2_call(agg2a, agg2b, h1, m2_root, m2_bias.reshape(1, 64), gcn_W)

    agg3 = _sc_gather_scatter(hw, src2, dsts, z32)
    i_pad = jnp.concatenate(
        [i, jnp.full((NPAD - N,), N_GRAPHS, jnp.int32)]).reshape(NPAD // BN, 1, BN)
    return _pool_call(agg3, i_pad, gcn_b.reshape(1, 32))


# speedup vs baseline: 2.6601x; 1.5440x over previous
import functools

import jax
import jax.numpy as jnp
from jax import lax
from jax.experimental import pallas as pl
from jax.experimental.pallas import tpu as pltpu
from jax.experimental.pallas import tpu_sc as plsc

N = 20000
E = 50000
F_IN = 30
D_E = 16
N_GRAPHS = 512

T_E = 512
EPAD = 53248
BN = 2048
NPAD = 20480

NC = 2
NS = 16
NW = NC * NS
CHUNK = 128
CH = EPAD // (NW * CHUNK)
PERW = CH * CHUNK
HALF = NPAD // 2
IMG = HALF + 128
SSTRIPE = IMG // NS
DSTRIPE = HALF // NS


def _relu(v):
    return jnp.maximum(v, 0.0)


def _sc_mesh():
    return plsc.VectorSubcoreMesh(core_axis_name="c", subcore_axis_name="s")


def _sc_gather(table, idx2):
    @functools.partial(
        pl.kernel,
        out_type=jax.ShapeDtypeStruct((EPAD, 32), jnp.float32),
        mesh=_sc_mesh(),
        compiler_params=pltpu.CompilerParams(use_tc_tiling_on_sc=False),
        scratch_types=[
            pltpu.VMEM((CH, CHUNK), jnp.int32),
            pltpu.VMEM((PERW, 32), jnp.float32),
            pltpu.SemaphoreType.DMA,
        ],
    )
    def k(table_hbm, idx_hbm, out_hbm, idx_v, rows_v, sem):
        wid = lax.axis_index("s") * NC + lax.axis_index("c")
        pltpu.sync_copy(idx_hbm.at[wid], idx_v)
        cps = [
            pltpu.async_copy(table_hbm.at[idx_v.at[j]],
                             rows_v.at[pl.ds(j * CHUNK, CHUNK)], sem)
            for j in range(CH)
        ]
        for d in cps:
            d.wait()
        pltpu.sync_copy(rows_v, out_hbm.at[pl.ds(wid * PERW, PERW)])

    return k(table, idx2)


def _sc_scatter(msg, dsts, zrows, c):
    @functools.partial(
        pl.kernel,
        out_type=jax.ShapeDtypeStruct((NPAD, c), jnp.float32),
        mesh=_sc_mesh(),
        compiler_params=pltpu.CompilerParams(use_tc_tiling_on_sc=False),
        scratch_types=[
            pltpu.VMEM((CH, CHUNK), jnp.int32),
            pltpu.VMEM((PERW, c), jnp.float32),
            pltpu.VMEM_SHARED((IMG, c), jnp.float32),
        ],
    )
    def k(msg_hbm, dst_hbm, z_hbm, out_hbm, dst_v, msg_v, shared):
        cc = lax.axis_index("c")
        s = lax.axis_index("s")
        pltpu.sync_copy(z_hbm.at[pl.ds(s * SSTRIPE, SSTRIPE)],
                        shared.at[pl.ds(s * SSTRIPE, SSTRIPE)])
        plsc.subcore_barrier()
        for half in range(2):
            w = s * 2 + half
            pltpu.sync_copy(dst_hbm.at[cc].at[w], dst_v)
            pltpu.sync_copy(msg_hbm.at[pl.ds(w * PERW, PERW)], msg_v)
            for j in range(CH):
                pltpu.sync_copy(msg_v.at[pl.ds(j * CHUNK, CHUNK)],
                                shared.at[dst_v.at[j]], add=True)
        plsc.subcore_barrier()
        pltpu.sync_copy(shared.at[pl.ds(s * DSTRIPE, DSTRIPE)],
                        out_hbm.at[pl.ds(cc * HALF + s * DSTRIPE, DSTRIPE)])

    return k(msg, dsts, zrows)


def _sc_gather_scatter(hw, src2, dsts, zrows):
    @functools.partial(
        pl.kernel,
        out_type=jax.ShapeDtypeStruct((NPAD, 32), jnp.float32),
        mesh=_sc_mesh(),
        compiler_params=pltpu.CompilerParams(use_tc_tiling_on_sc=False),
        scratch_types=[
            pltpu.VMEM((CH, CHUNK), jnp.int32),
            pltpu.VMEM((CH, CHUNK), jnp.int32),
            pltpu.VMEM((PERW, 32), jnp.float32),
            pltpu.VMEM_SHARED((IMG, 32), jnp.float32),
            pltpu.SemaphoreType.DMA,
        ],
    )
    def k(hw_hbm, src_hbm, dst_hbm, z_hbm, out_hbm, src_v, dst_v, rows_v,
          shared, sem):
        cc = lax.axis_index("c")
        s = lax.axis_index("s")
        pltpu.sync_copy(z_hbm.at[pl.ds(s * SSTRIPE, SSTRIPE)],
                        shared.at[pl.ds(s * SSTRIPE, SSTRIPE)])
        plsc.subcore_barrier()
        for half in range(2):
            w = s * 2 + half
            pltpu.sync_copy(src_hbm.at[w], src_v)
            pltpu.sync_copy(dst_hbm.at[cc].at[w], dst_v)
            cps = [
                pltpu.async_copy(hw_hbm.at[src_v.at[j]],
                                 rows_v.at[pl.ds(j * CHUNK, CHUNK)], sem)
                for j in range(CH)
            ]
            for d in cps:
                d.wait()
            for j in range(CH):
                pltpu.sync_copy(rows_v.at[pl.ds(j * CHUNK, CHUNK)],
                                shared.at[dst_v.at[j]], add=True)
        plsc.subcore_barrier()
        pltpu.sync_copy(shared.at[pl.ds(s * DSTRIPE, DSTRIPE)],
                        out_hbm.at[pl.ds(cc * HALF + s * DSTRIPE, DSTRIPE)])

    return k(hw, src2, dsts, zrows)


def _edge_body(e_ref, xg_ref, w1_ref, b1_ref, w2_ref, b2_ref, rx_ref, rh_ref,
               m_ref, *o_refs, c):
    h = _relu(jnp.dot(e_ref[...], w1_ref[...],
                      preferred_element_type=jnp.float32) + b1_ref[...])
    h = _relu(jnp.dot(h, w2_ref[...],
                      preferred_element_type=jnp.float32) + b2_ref[...])
    lane = jax.lax.broadcasted_iota(jnp.int32, h.shape, 1)
    h = jnp.where(lane == 30, 1.0, h)
    xt = jnp.dot(xg_ref[...], rx_ref[...], preferred_element_type=jnp.float32)
    hr = jnp.dot(h, rh_ref[...], preferred_element_type=jnp.float32)
    p = xt * hr
    msg = jnp.dot(p, m_ref[...], preferred_element_type=jnp.float32)
    row = jax.lax.broadcasted_iota(jnp.int32, msg.shape, 0) + pl.program_id(0) * T_E
    msg = jnp.where(row < E, msg, 0.0)
    for g, o_ref in enumerate(o_refs):
        o_ref[...] = msg[:, g * 32:(g + 1) * 32]


def _edge_call(e_pad, xg, w1p, b1p, w2p, b2p, rx, rh, mp, c):
    grid = EPAD // T_E
    return pl.pallas_call(
        functools.partial(_edge_body, c=c),
        grid=(grid,),
        in_specs=[
            pl.BlockSpec((T_E, D_E), lambda n: (n, 0)),
            pl.BlockSpec((T_E, 32), lambda n: (n, 0)),
            pl.BlockSpec((D_E, 32), lambda n: (0, 0)),
            pl.BlockSpec((1, 32), lambda n: (0, 0)),
            pl.BlockSpec((32, 32), lambda n: (0, 0)),
            pl.BlockSpec((1, 32), lambda n: (0, 0)),
            pl.BlockSpec((32, 32 * 32), lambda n: (0, 0)),
            pl.BlockSpec((32, 32 * 32), lambda n: (0, 0)),
            pl.BlockSpec((32 * 32, c), lambda n: (0, 0)),
        ],
        out_specs=[pl.BlockSpec((T_E, 32), lambda n: (n, 0))] * (c // 32),
        out_shape=[jax.ShapeDtypeStruct((EPAD, 32), jnp.float32)] * (c // 32),
    )(e_pad, xg, w1p, b1p, w2p, b2p, rx, rh, mp)


def _node1_body(agg_ref, x_ref, root_ref, bias_ref, o_ref):
    o_ref[...] = _relu(
        agg_ref[...]
        + jnp.dot(x_ref[...], root_ref[...], preferred_element_type=jnp.float32)
        + bias_ref[...])


def _node1_call(agg, x, root, bias):
    grid = NPAD // BN
    return pl.pallas_call(
        _node1_body,
        grid=(grid,),
        in_specs=[
            pl.BlockSpec((BN, 32), lambda n: (n, 0)),
            pl.BlockSpec((BN, F_IN), lambda n: (n, 0)),
            pl.BlockSpec((F_IN, 32), lambda n: (0, 0)),
            pl.BlockSpec((1, 32), lambda n: (0, 0)),
        ],
        out_specs=pl.BlockSpec((BN, 32), lambda n: (n, 0)),
        out_shape=jax.ShapeDtypeStruct((NPAD, 32), jnp.float32),
    )(agg, x, root, bias)


def _node2_body(agga_ref, aggb_ref, h1_ref, root_ref, bias_ref, gw_ref, o_ref):
    rt = jnp.dot(h1_ref[...], root_ref[...], preferred_element_type=jnp.float32)
    h2a = _relu(agga_ref[...] + rt[:, :32] + bias_ref[...][:, :32])
    h2b = _relu(aggb_ref[...] + rt[:, 32:] + bias_ref[...][:, 32:])
    o_ref[...] = (
        jnp.dot(h2a, gw_ref[...][:32], preferred_element_type=jnp.float32)
        + jnp.dot(h2b, gw_ref[...][32:], preferred_element_type=jnp.float32))


def _node2_call(agga, aggb, h1, root, bias, gw):
    grid = NPAD // BN
    return pl.pallas_call(
        _node2_body,
        grid=(grid,),
        in_specs=[
            pl.BlockSpec((BN, 32), lambda n: (n, 0)),
            pl.BlockSpec((BN, 32), lambda n: (n, 0)),
            pl.BlockSpec((BN, 32), lambda n: (n, 0)),
            pl.BlockSpec((32, 64), lambda n: (0, 0)),
            pl.BlockSpec((1, 64), lambda n: (0, 0)),
            pl.BlockSpec((64, 32), lambda n: (0, 0)),
        ],
        out_specs=pl.BlockSpec((BN, 32), lambda n: (n, 0)),
        out_shape=jax.ShapeDtypeStruct((NPAD, 32), jnp.float32),
    )(agga, aggb, h1, root, bias, gw)


def _pool_body(agg_ref, i_ref, b_ref, o_ref):
    n = pl.program_id(0)
    h3 = _relu(agg_ref[...] + b_ref[...])
    iv = i_ref[0]
    onehot = jnp.where(
        jax.lax.broadcasted_iota(jnp.int32, (N_GRAPHS, BN), 0) == iv, 1.0, 0.0)
    contrib = jnp.dot(onehot, h3, preferred_element_type=jnp.float32)

    @pl.when(n == 0)
    def _():
        o_ref[...] = jnp.zeros_like(o_ref)

    o_ref[...] += contrib


def _pool_call(agg3, i3, gcn_b):
    grid = NPAD // BN
    return pl.pallas_call(
        _pool_body,
        grid=(grid,),
        in_specs=[
            pl.BlockSpec((BN, 32), lambda n: (n, 0)),
            pl.BlockSpec((1, 1, BN), lambda n: (n, 0, 0)),
            pl.BlockSpec((1, 32), lambda n: (0, 0)),
        ],
        out_specs=pl.BlockSpec((N_GRAPHS, 32), lambda n: (0, 0)),
        out_shape=jax.ShapeDtypeStruct((N_GRAPHS, 32), jnp.float32),
    )(agg3, i3, gcn_b)


def _make_m(wk, bk, f, c):
    wk_aug = jnp.concatenate(
        [wk, bk[None, :], jnp.zeros((1, f * c), jnp.float32)], axis=0)
    a = wk_aug.reshape(32, f, c)
    if f < 32:
        a = jnp.pad(a, ((0, 0), (0, 32 - f), (0, 0)))
    return a.reshape(32 * 32, c)


def _pad2(w, rows, cols):
    return jnp.pad(w, ((0, rows - w.shape[0]), (0, cols - w.shape[1])))


def kernel(x, edge_index, e, i, m1_W1, m1_b1, m1_W2, m1_b2, m1_Wk, m1_bk,
           m1_root, m1_bias, m2_W1, m2_b1, m2_W2, m2_b2, m2_Wk, m2_bk,
           m2_root, m2_bias, gcn_W, gcn_b):
    src = edge_index[0]
    dst = edge_index[1]
    src2 = jnp.concatenate(
        [src, jnp.zeros((EPAD - E,), jnp.int32)]).reshape(NW, CH, CHUNK)
    dpad = jnp.concatenate([dst, jnp.full((EPAD - E,), NPAD - 1, jnp.int32)])
    d0 = jnp.where(dpad < HALF, dpad, HALF)
    d1r = dpad - HALF
    d1 = jnp.where((d1r >= 0) & (d1r < HALF), d1r, HALF)
    dsts = jnp.stack([d0, d1]).reshape(NC, NW, CH, CHUNK)
    z32 = jnp.zeros((IMG, 32), jnp.float32)

    e_pad = jnp.pad(e, ((0, EPAD - E), (0, 0)))
    x32 = jnp.pad(x, ((0, 0), (0, 32 - F_IN)))

    m1w1 = _pad2(m1_W1, D_E, 32)
    m1b1 = jnp.pad(m1_b1, (0, 2)).reshape(1, 32)
    m1w2 = _pad2(m1_W2, 32, 32)
    m1b2 = jnp.pad(m1_b2, (0, 2)).reshape(1, 32)
    m1m = _make_m(m1_Wk, m1_bk, F_IN, 32)
    m2w1 = _pad2(m2_W1, D_E, 32)
    m2b1 = jnp.pad(m2_b1, (0, 2)).reshape(1, 32)
    m2w2 = _pad2(m2_W2, 32, 32)
    m2b2 = jnp.pad(m2_b2, (0, 2)).reshape(1, 32)
    m2m = _make_m(m2_Wk, m2_bk, 32, 64)

    eye = jnp.eye(32, dtype=jnp.float32)
    rx = jnp.kron(jnp.ones((1, 32), jnp.float32), eye)
    rh = jnp.kron(eye, jnp.ones((1, 32), jnp.float32))

    xg = _sc_gather(x32, src2)
    (msg1,) = _edge_call(e_pad, xg, m1w1, m1b1, m1w2, m1b2, rx, rh, m1m, 32)
    agg1 = _sc_scatter(msg1, dsts, z32, 32)
    x_n = jnp.pad(x, ((0, NPAD - N), (0, 0)))
    h1 = _node1_call(agg1, x_n, m1_root, m1_bias.reshape(1, 32))

    h1g = _sc_gather(h1, src2)
    msg2a, msg2b = _edge_call(e_pad, h1g, m2w1, m2b1, m2w2, m2b2, rx, rh, m2m, 64)
    agg2a = _sc_scatter(msg2a, dsts, z32, 32)
    agg2b = _sc_scatter(msg2b, dsts, z32, 32)
    hw = _node2_call(agg2a, agg2b, h1, m2_root, m2_bias.reshape(1, 64), gcn_W)

    agg3 = _sc_gather_scatter(hw, src2, dsts, z32)
    i_pad = jnp.concatenate(
        [i, jnp.full((NPAD - N,), N_GRAPHS, jnp.int32)]).reshape(NPAD // BN, 1, BN)
    return _pool_call(agg3, i_pad, gcn_b.reshape(1, 32))


# speedup vs baseline: 2.6658x; 1.0021x over previous
import functools

import jax
import jax.numpy as jnp
from jax import lax
from jax.experimental import pallas as pl
from jax.experimental.pallas import tpu as pltpu
from jax.experimental.pallas import tpu_sc as plsc

N = 20000
E = 50000
F_IN = 30
D_E = 16
N_GRAPHS = 512

T_E = 512
EPAD = 53248
BN = 2048
NPAD = 20480

NC = 2
NS = 16
NW = NC * NS
CHUNK = 128
CH = EPAD // (NW * CHUNK)
PERW = CH * CHUNK
HALF = NPAD // 2
IMG = HALF + 128
SSTRIPE = IMG // NS
DSTRIPE = HALF // NS


def _relu(v):
    return jnp.maximum(v, 0.0)


def _sc_mesh():
    return plsc.VectorSubcoreMesh(core_axis_name="c", subcore_axis_name="s")


def _sc_gather(table, idx2):
    @functools.partial(
        pl.kernel,
        out_type=jax.ShapeDtypeStruct((EPAD, 32), jnp.float32),
        mesh=_sc_mesh(),
        compiler_params=pltpu.CompilerParams(use_tc_tiling_on_sc=False),
        scratch_types=[
            pltpu.VMEM((PERW,), jnp.int32),
            pltpu.VMEM((PERW, 32), jnp.float32),
            pltpu.SemaphoreType.DMA,
        ],
    )
    def k(table_hbm, idx_hbm, out_hbm, idx_v, rows_v, sem):
        wid = lax.axis_index("s") * NC + lax.axis_index("c")
        pltpu.sync_copy(idx_hbm.at[wid], idx_v)
        pltpu.async_copy(table_hbm.at[idx_v], rows_v, sem).wait()
        pltpu.sync_copy(rows_v, out_hbm.at[pl.ds(wid * PERW, PERW)])

    return k(table, idx2)


def _sc_scatter(msg, dsts, zrows, c):
    @functools.partial(
        pl.kernel,
        out_type=jax.ShapeDtypeStruct((NPAD, c), jnp.float32),
        mesh=_sc_mesh(),
        compiler_params=pltpu.CompilerParams(use_tc_tiling_on_sc=False),
        scratch_types=[
            pltpu.VMEM((CH, CHUNK), jnp.int32),
            pltpu.VMEM((PERW, c), jnp.float32),
            pltpu.VMEM_SHARED((IMG, c), jnp.float32),
        ],
    )
    def k(msg_hbm, dst_hbm, z_hbm, out_hbm, dst_v, msg_v, shared):
        cc = lax.axis_index("c")
        s = lax.axis_index("s")
        pltpu.sync_copy(z_hbm.at[pl.ds(s * SSTRIPE, SSTRIPE)],
                        shared.at[pl.ds(s * SSTRIPE, SSTRIPE)])
        plsc.subcore_barrier()
        for half in range(2):
            w = s * 2 + half
            pltpu.sync_copy(dst_hbm.at[cc].at[w], dst_v)
            pltpu.sync_copy(msg_hbm.at[pl.ds(w * PERW, PERW)], msg_v)
            for j in range(CH):
                pltpu.sync_copy(msg_v.at[pl.ds(j * CHUNK, CHUNK)],
                                shared.at[dst_v.at[j]], add=True)
        plsc.subcore_barrier()
        pltpu.sync_copy(shared.at[pl.ds(s * DSTRIPE, DSTRIPE)],
                        out_hbm.at[pl.ds(cc * HALF + s * DSTRIPE, DSTRIPE)])

    return k(msg, dsts, zrows)


def _sc_gather_scatter(hw, src2, dsts, zrows):
    @functools.partial(
        pl.kernel,
        out_type=jax.ShapeDtypeStruct((NPAD, 32), jnp.float32),
        mesh=_sc_mesh(),
        compiler_params=pltpu.CompilerParams(use_tc_tiling_on_sc=False),
        scratch_types=[
            pltpu.VMEM((PERW,), jnp.int32),
            pltpu.VMEM((CH, CHUNK), jnp.int32),
            pltpu.VMEM((PERW, 32), jnp.float32),
            pltpu.VMEM_SHARED((IMG, 32), jnp.float32),
            pltpu.SemaphoreType.DMA,
        ],
    )
    def k(hw_hbm, src_hbm, dst_hbm, z_hbm, out_hbm, src_v, dst_v, rows_v,
          shared, sem):
        cc = lax.axis_index("c")
        s = lax.axis_index("s")
        pltpu.sync_copy(z_hbm.at[pl.ds(s * SSTRIPE, SSTRIPE)],
                        shared.at[pl.ds(s * SSTRIPE, SSTRIPE)])
        plsc.subcore_barrier()
        for half in range(2):
            w = s * 2 + half
            pltpu.sync_copy(src_hbm.at[w], src_v)
            pltpu.sync_copy(dst_hbm.at[cc].at[w], dst_v)
            pltpu.async_copy(hw_hbm.at[src_v], rows_v, sem).wait()
            for j in range(CH):
                pltpu.sync_copy(rows_v.at[pl.ds(j * CHUNK, CHUNK)],
                                shared.at[dst_v.at[j]], add=True)
        plsc.subcore_barrier()
        pltpu.sync_copy(shared.at[pl.ds(s * DSTRIPE, DSTRIPE)],
                        out_hbm.at[pl.ds(cc * HALF + s * DSTRIPE, DSTRIPE)])

    return k(hw, src2, dsts, zrows)


def _edge_body(e_ref, xg_ref, w1_ref, b1_ref, w2_ref, b2_ref, rx_ref, rh_ref,
               m_ref, *o_refs, c):
    h = _relu(jnp.dot(e_ref[...], w1_ref[...],
                      preferred_element_type=jnp.float32) + b1_ref[...])
    h = _relu(jnp.dot(h, w2_ref[...],
                      preferred_element_type=jnp.float32) + b2_ref[...])
    lane = jax.lax.broadcasted_iota(jnp.int32, h.shape, 1)
    h = jnp.where(lane == 30, 1.0, h)
    xt = jnp.dot(xg_ref[...], rx_ref[...], preferred_element_type=jnp.float32)
    hr = jnp.dot(h, rh_ref[...], preferred_element_type=jnp.float32)
    p = xt * hr
    msg = jnp.dot(p, m_ref[...], preferred_element_type=jnp.float32)
    row = jax.lax.broadcasted_iota(jnp.int32, msg.shape, 0) + pl.program_id(0) * T_E
    msg = jnp.where(row < E, msg, 0.0)
    for g, o_ref in enumerate(o_refs):
        o_ref[...] = msg[:, g * 32:(g + 1) * 32]


def _edge_call(e_pad, xg, w1p, b1p, w2p, b2p, rx, rh, mp, c):
    grid = EPAD // T_E
    return pl.pallas_call(
        functools.partial(_edge_body, c=c),
        grid=(grid,),
        in_specs=[
            pl.BlockSpec((T_E, D_E), lambda n: (n, 0)),
            pl.BlockSpec((T_E, 32), lambda n: (n, 0)),
            pl.BlockSpec((D_E, 32), lambda n: (0, 0)),
            pl.BlockSpec((1, 32), lambda n: (0, 0)),
            pl.BlockSpec((32, 32), lambda n: (0, 0)),
            pl.BlockSpec((1, 32), lambda n: (0, 0)),
            pl.BlockSpec((32, 32 * 32), lambda n: (0, 0)),
            pl.BlockSpec((32, 32 * 32), lambda n: (0, 0)),
            pl.BlockSpec((32 * 32, c), lambda n: (0, 0)),
        ],
        out_specs=[pl.BlockSpec((T_E, 32), lambda n: (n, 0))] * (c // 32),
        out_shape=[jax.ShapeDtypeStruct((EPAD, 32), jnp.float32)] * (c // 32),
    )(e_pad, xg, w1p, b1p, w2p, b2p, rx, rh, mp)


def _node1_body(agg_ref, x_ref, root_ref, bias_ref, o_ref):
    o_ref[...] = _relu(
        agg_ref[...]
        + jnp.dot(x_ref[...], root_ref[...], preferred_element_type=jnp.float32)
        + bias_ref[...])


def _node1_call(agg, x, root, bias):
    grid = NPAD // BN
    return pl.pallas_call(
        _node1_body,
        grid=(grid,),
        in_specs=[
            pl.BlockSpec((BN, 32), lambda n: (n, 0)),
            pl.BlockSpec((BN, F_IN), lambda n: (n, 0)),
            pl.BlockSpec((F_IN, 32), lambda n: (0, 0)),
            pl.BlockSpec((1, 32), lambda n: (0, 0)),
        ],
        out_specs=pl.BlockSpec((BN, 32), lambda n: (n, 0)),
        out_shape=jax.ShapeDtypeStruct((NPAD, 32), jnp.float32),
    )(agg, x, root, bias)


def _node2_body(agga_ref, aggb_ref, h1_ref, root_ref, bias_ref, gw_ref, o_ref):
    rt = jnp.dot(h1_ref[...], root_ref[...], preferred_element_type=jnp.float32)
    h2a = _relu(agga_ref[...] + rt[:, :32] + bias_ref[...][:, :32])
    h2b = _relu(aggb_ref[...] + rt[:, 32:] + bias_ref[...][:, 32:])
    o_ref[...] = (
        jnp.dot(h2a, gw_ref[...][:32], preferred_element_type=jnp.float32)
        + jnp.dot(h2b, gw_ref[...][32:], preferred_element_type=jnp.float32))


def _node2_call(agga, aggb, h1, root, bias, gw):
    grid = NPAD // BN
    return pl.pallas_call(
        _node2_body,
        grid=(grid,),
        in_specs=[
            pl.BlockSpec((BN, 32), lambda n: (n, 0)),
            pl.BlockSpec((BN, 32), lambda n: (n, 0)),
            pl.BlockSpec((BN, 32), lambda n: (n, 0)),
            pl.BlockSpec((32, 64), lambda n: (0, 0)),
            pl.BlockSpec((1, 64), lambda n: (0, 0)),
            pl.BlockSpec((64, 32), lambda n: (0, 0)),
        ],
        out_specs=pl.BlockSpec((BN, 32), lambda n: (n, 0)),
        out_shape=jax.ShapeDtypeStruct((NPAD, 32), jnp.float32),
    )(agga, aggb, h1, root, bias, gw)


def _pool_body(agg_ref, i_ref, b_ref, o_ref):
    n = pl.program_id(0)
    h3 = _relu(agg_ref[...] + b_ref[...])
    iv = i_ref[0]
    onehot = jnp.where(
        jax.lax.broadcasted_iota(jnp.int32, (N_GRAPHS, BN), 0) == iv, 1.0, 0.0)
    contrib = jnp.dot(onehot, h3, preferred_element_type=jnp.float32)

    @pl.when(n == 0)
    def _():
        o_ref[...] = jnp.zeros_like(o_ref)

    o_ref[...] += contrib


def _pool_call(agg3, i3, gcn_b):
    grid = NPAD // BN
    return pl.pallas_call(
        _pool_body,
        grid=(grid,),
        in_specs=[
            pl.BlockSpec((BN, 32), lambda n: (n, 0)),
            pl.BlockSpec((1, 1, BN), lambda n: (n, 0, 0)),
            pl.BlockSpec((1, 32), lambda n: (0, 0)),
        ],
        out_specs=pl.BlockSpec((N_GRAPHS, 32), lambda n: (0, 0)),
        out_shape=jax.ShapeDtypeStruct((N_GRAPHS, 32), jnp.float32),
    )(agg3, i3, gcn_b)


def _make_m(wk, bk, f, c):
    wk_aug = jnp.concatenate(
        [wk, bk[None, :], jnp.zeros((1, f * c), jnp.float32)], axis=0)
    a = wk_aug.reshape(32, f, c)
    if f < 32:
        a = jnp.pad(a, ((0, 0), (0, 32 - f), (0, 0)))
    return a.reshape(32 * 32, c)


def _pad2(w, rows, cols):
    return jnp.pad(w, ((0, rows - w.shape[0]), (0, cols - w.shape[1])))


def kernel(x, edge_index, e, i, m1_W1, m1_b1, m1_W2, m1_b2, m1_Wk, m1_bk,
           m1_root, m1_bias, m2_W1, m2_b1, m2_W2, m2_b2, m2_Wk, m2_bk,
           m2_root, m2_bias, gcn_W, gcn_b):
    src = edge_index[0]
    dst = edge_index[1]
    src2 = jnp.concatenate(
        [src, jnp.zeros((EPAD - E,), jnp.int32)]).reshape(NW, PERW)
    dpad = jnp.concatenate([dst, jnp.full((EPAD - E,), NPAD - 1, jnp.int32)])
    d0 = jnp.where(dpad < HALF, dpad, HALF)
    d1r = dpad - HALF
    d1 = jnp.where((d1r >= 0) & (d1r < HALF), d1r, HALF)
    dsts = jnp.stack([d0, d1]).reshape(NC, NW, CH, CHUNK)
    z32 = jnp.zeros((IMG, 32), jnp.float32)

    e_pad = jnp.pad(e, ((0, EPAD - E), (0, 0)))
    x32 = jnp.pad(x, ((0, 0), (0, 32 - F_IN)))

    m1w1 = _pad2(m1_W1, D_E, 32)
    m1b1 = jnp.pad(m1_b1, (0, 2)).reshape(1, 32)
    m1w2 = _pad2(m1_W2, 32, 32)
    m1b2 = jnp.pad(m1_b2, (0, 2)).reshape(1, 32)
    m1m = _make_m(m1_Wk, m1_bk, F_IN, 32)
    m2w1 = _pad2(m2_W1, D_E, 32)
    m2b1 = jnp.pad(m2_b1, (0, 2)).reshape(1, 32)
    m2w2 = _pad2(m2_W2, 32, 32)
    m2b2 = jnp.pad(m2_b2, (0, 2)).reshape(1, 32)
    m2m = _make_m(m2_Wk, m2_bk, 32, 64)

    eye = jnp.eye(32, dtype=jnp.float32)
    rx = jnp.kron(jnp.ones((1, 32), jnp.float32), eye)
    rh = jnp.kron(eye, jnp.ones((1, 32), jnp.float32))

    xg = _sc_gather(x32, src2)
    (msg1,) = _edge_call(e_pad, xg, m1w1, m1b1, m1w2, m1b2, rx, rh, m1m, 32)
    agg1 = _sc_scatter(msg1, dsts, z32, 32)
    x_n = jnp.pad(x, ((0, NPAD - N), (0, 0)))
    h1 = _node1_call(agg1, x_n, m1_root, m1_bias.reshape(1, 32))

    h1g = _sc_gather(h1, src2)
    msg2a, msg2b = _edge_call(e_pad, h1g, m2w1, m2b1, m2w2, m2b2, rx, rh, m2m, 64)
    agg2a = _sc_scatter(msg2a, dsts, z32, 32)
    agg2b = _sc_scatter(msg2b, dsts, z32, 32)
    hw = _node2_call(agg2a, agg2b, h1, m2_root, m2_bias.reshape(1, 64), gcn_W)

    agg3 = _sc_gather_scatter(hw, src2, dsts, z32)
    i_pad = jnp.concatenate(
        [i, jnp.full((NPAD - N,), N_GRAPHS, jnp.int32)]).reshape(NPAD // BN, 1, BN)
    return _pool_call(agg3, i_pad, gcn_b.reshape(1, 32))


# speedup vs baseline: 3.0953x; 1.1611x over previous
import functools

import jax
import jax.numpy as jnp
from jax import lax
from jax.experimental import pallas as pl
from jax.experimental.pallas import tpu as pltpu
from jax.experimental.pallas import tpu_sc as plsc

N = 20000
E = 50000
F_IN = 30
D_E = 16
N_GRAPHS = 512

T_E = 512
EPAD = 53248
BN = 2048
NPAD = 20480

NC = 2
NS = 16
NW = NC * NS
CHUNK = 128
CH = EPAD // (NW * CHUNK)
PERW = CH * CHUNK
HALF = NPAD // 2
IMG = HALF + 128
SSTRIPE = IMG // NS
DSTRIPE = HALF // NS


def _relu(v):
    return jnp.maximum(v, 0.0)


def _sc_mesh():
    return plsc.VectorSubcoreMesh(core_axis_name="c", subcore_axis_name="s")


def _sc_gather(table, idx2):
    @functools.partial(
        pl.kernel,
        out_type=jax.ShapeDtypeStruct((EPAD, 32), jnp.bfloat16),
        mesh=_sc_mesh(),
        compiler_params=pltpu.CompilerParams(use_tc_tiling_on_sc=False),
        scratch_types=[
            pltpu.VMEM((PERW,), jnp.int32),
            pltpu.VMEM((PERW, 32), jnp.bfloat16),
            pltpu.SemaphoreType.DMA,
        ],
    )
    def k(table_hbm, idx_hbm, out_hbm, idx_v, rows_v, sem):
        wid = lax.axis_index("s") * NC + lax.axis_index("c")
        pltpu.sync_copy(idx_hbm.at[wid], idx_v)
        pltpu.async_copy(table_hbm.at[idx_v], rows_v, sem).wait()
        pltpu.sync_copy(rows_v, out_hbm.at[pl.ds(wid * PERW, PERW)])

    return k(table, idx2)


def _sc_scatter(msg, dsts, zrows, c):
    @functools.partial(
        pl.kernel,
        out_type=jax.ShapeDtypeStruct((NPAD, c), jnp.bfloat16),
        mesh=_sc_mesh(),
        compiler_params=pltpu.CompilerParams(use_tc_tiling_on_sc=False),
        scratch_types=[
            pltpu.VMEM((CH, CHUNK), jnp.int32),
            pltpu.VMEM((PERW, c), jnp.bfloat16),
            pltpu.VMEM_SHARED((IMG, c), jnp.bfloat16),
        ],
    )
    def k(msg_hbm, dst_hbm, z_hbm, out_hbm, dst_v, msg_v, shared):
        cc = lax.axis_index("c")
        s = lax.axis_index("s")
        pltpu.sync_copy(z_hbm.at[pl.ds(s * SSTRIPE, SSTRIPE)],
                        shared.at[pl.ds(s * SSTRIPE, SSTRIPE)])
        plsc.subcore_barrier()
        for half in range(2):
            w = s * 2 + half
            pltpu.sync_copy(dst_hbm.at[cc].at[w], dst_v)
            pltpu.sync_copy(msg_hbm.at[pl.ds(w * PERW, PERW)], msg_v)
            for j in range(CH):
                pltpu.sync_copy(msg_v.at[pl.ds(j * CHUNK, CHUNK)],
                                shared.at[dst_v.at[j]], add=True)
        plsc.subcore_barrier()
        pltpu.sync_copy(shared.at[pl.ds(s * DSTRIPE, DSTRIPE)],
                        out_hbm.at[pl.ds(cc * HALF + s * DSTRIPE, DSTRIPE)])

    return k(msg, dsts, zrows)


def _sc_gather_scatter(hw, src2, dsts, zrows):
    @functools.partial(
        pl.kernel,
        out_type=jax.ShapeDtypeStruct((NPAD, 32), jnp.bfloat16),
        mesh=_sc_mesh(),
        compiler_params=pltpu.CompilerParams(use_tc_tiling_on_sc=False),
        scratch_types=[
            pltpu.VMEM((PERW,), jnp.int32),
            pltpu.VMEM((CH, CHUNK), jnp.int32),
            pltpu.VMEM((PERW, 32), jnp.bfloat16),
            pltpu.VMEM_SHARED((IMG, 32), jnp.bfloat16),
            pltpu.SemaphoreType.DMA,
        ],
    )
    def k(hw_hbm, src_hbm, dst_hbm, z_hbm, out_hbm, src_v, dst_v, rows_v,
          shared, sem):
        cc = lax.axis_index("c")
        s = lax.axis_index("s")
        pltpu.sync_copy(z_hbm.at[pl.ds(s * SSTRIPE, SSTRIPE)],
                        shared.at[pl.ds(s * SSTRIPE, SSTRIPE)])
        plsc.subcore_barrier()
        for half in range(2):
            w = s * 2 + half
            pltpu.sync_copy(src_hbm.at[w], src_v)
            pltpu.sync_copy(dst_hbm.at[cc].at[w], dst_v)
            pltpu.async_copy(hw_hbm.at[src_v], rows_v, sem).wait()
            for j in range(CH):
                pltpu.sync_copy(rows_v.at[pl.ds(j * CHUNK, CHUNK)],
                                shared.at[dst_v.at[j]], add=True)
        plsc.subcore_barrier()
        pltpu.sync_copy(shared.at[pl.ds(s * DSTRIPE, DSTRIPE)],
                        out_hbm.at[pl.ds(cc * HALF + s * DSTRIPE, DSTRIPE)])

    return k(hw, src2, dsts, zrows)


def _edge_body(e_ref, xg_ref, w1_ref, b1_ref, w2_ref, b2_ref, rx_ref, rh_ref,
               m_ref, *o_refs, c):
    h = _relu(jnp.dot(e_ref[...], w1_ref[...],
                      preferred_element_type=jnp.float32) + b1_ref[...])
    h = _relu(jnp.dot(h, w2_ref[...],
                      preferred_element_type=jnp.float32) + b2_ref[...])
    lane = jax.lax.broadcasted_iota(jnp.int32, h.shape, 1)
    h = jnp.where(lane == 30, 1.0, h)
    xt = jnp.dot(xg_ref[...], rx_ref[...],
                 preferred_element_type=jnp.float32)
    hr = jnp.dot(h.astype(jnp.bfloat16), rh_ref[...],
                 preferred_element_type=jnp.float32)
    p = (xt * hr).astype(jnp.bfloat16)
    msg = jnp.dot(p, m_ref[...], preferred_element_type=jnp.float32)
    row = jax.lax.broadcasted_iota(jnp.int32, msg.shape, 0) + pl.program_id(0) * T_E
    msg = jnp.where(row < E, msg, 0.0)
    for g, o_ref in enumerate(o_refs):
        o_ref[...] = msg[:, g * 32:(g + 1) * 32].astype(jnp.bfloat16)


def _edge_call(e_pad, xg, w1p, b1p, w2p, b2p, rx, rh, mp, c):
    grid = EPAD // T_E
    return pl.pallas_call(
        functools.partial(_edge_body, c=c),
        grid=(grid,),
        in_specs=[
            pl.BlockSpec((T_E, D_E), lambda n: (n, 0)),
            pl.BlockSpec((T_E, 32), lambda n: (n, 0)),
            pl.BlockSpec((D_E, 32), lambda n: (0, 0)),
            pl.BlockSpec((1, 32), lambda n: (0, 0)),
            pl.BlockSpec((32, 32), lambda n: (0, 0)),
            pl.BlockSpec((1, 32), lambda n: (0, 0)),
            pl.BlockSpec((32, 32 * 32), lambda n: (0, 0)),
            pl.BlockSpec((32, 32 * 32), lambda n: (0, 0)),
            pl.BlockSpec((32 * 32, c), lambda n: (0, 0)),
        ],
        out_specs=[pl.BlockSpec((T_E, 32), lambda n: (n, 0))] * (c // 32),
        out_shape=[jax.ShapeDtypeStruct((EPAD, 32), jnp.bfloat16)] * (c // 32),
    )(e_pad, xg, w1p, b1p, w2p, b2p, rx, rh, mp)


def _node1_body(agg_ref, x_ref, root_ref, bias_ref, o_ref):
    o_ref[...] = _relu(
        agg_ref[...].astype(jnp.float32)
        + jnp.dot(x_ref[...], root_ref[...], preferred_element_type=jnp.float32)
        + bias_ref[...]).astype(jnp.bfloat16)


def _node1_call(agg, x, root, bias):
    grid = NPAD // BN
    return pl.pallas_call(
        _node1_body,
        grid=(grid,),
        in_specs=[
            pl.BlockSpec((BN, 32), lambda n: (n, 0)),
            pl.BlockSpec((BN, F_IN), lambda n: (n, 0)),
            pl.BlockSpec((F_IN, 32), lambda n: (0, 0)),
            pl.BlockSpec((1, 32), lambda n: (0, 0)),
        ],
        out_specs=pl.BlockSpec((BN, 32), lambda n: (n, 0)),
        out_shape=jax.ShapeDtypeStruct((NPAD, 32), jnp.bfloat16),
    )(agg, x, root, bias)


def _node2_body(agga_ref, aggb_ref, h1_ref, root_ref, bias_ref, gw_ref, o_ref):
    rt = jnp.dot(h1_ref[...].astype(jnp.float32), root_ref[...],
                 preferred_element_type=jnp.float32)
    h2a = _relu(agga_ref[...].astype(jnp.float32) + rt[:, :32]
                + bias_ref[...][:, :32])
    h2b = _relu(aggb_ref[...].astype(jnp.float32) + rt[:, 32:]
                + bias_ref[...][:, 32:])
    o_ref[...] = (
        jnp.dot(h2a, gw_ref[...][:32], preferred_element_type=jnp.float32)
        + jnp.dot(h2b, gw_ref[...][32:], preferred_element_type=jnp.float32)
    ).astype(jnp.bfloat16)


def _node2_call(agga, aggb, h1, root, bias, gw):
    grid = NPAD // BN
    return pl.pallas_call(
        _node2_body,
        grid=(grid,),
        in_specs=[
            pl.BlockSpec((BN, 32), lambda n: (n, 0)),
            pl.BlockSpec((BN, 32), lambda n: (n, 0)),
            pl.BlockSpec((BN, 32), lambda n: (n, 0)),
            pl.BlockSpec((32, 64), lambda n: (0, 0)),
            pl.BlockSpec((1, 64), lambda n: (0, 0)),
            pl.BlockSpec((64, 32), lambda n: (0, 0)),
        ],
        out_specs=pl.BlockSpec((BN, 32), lambda n: (n, 0)),
        out_shape=jax.ShapeDtypeStruct((NPAD, 32), jnp.bfloat16),
    )(agga, aggb, h1, root, bias, gw)


def _pool_body(agg_ref, i_ref, b_ref, o_ref):
    n = pl.program_id(0)
    h3 = _relu(agg_ref[...].astype(jnp.float32) + b_ref[...])
    iv = i_ref[0]
    onehot = jnp.where(
        jax.lax.broadcasted_iota(jnp.int32, (N_GRAPHS, BN), 0) == iv, 1.0, 0.0)
    contrib = jnp.dot(onehot, h3, preferred_element_type=jnp.float32)

    @pl.when(n == 0)
    def _():
        o_ref[...] = jnp.zeros_like(o_ref)

    o_ref[...] += contrib


def _pool_call(agg3, i3, gcn_b):
    grid = NPAD // BN
    return pl.pallas_call(
        _pool_body,
        grid=(grid,),
        in_specs=[
            pl.BlockSpec((BN, 32), lambda n: (n, 0)),
            pl.BlockSpec((1, 1, BN), lambda n: (n, 0, 0)),
            pl.BlockSpec((1, 32), lambda n: (0, 0)),
        ],
        out_specs=pl.BlockSpec((N_GRAPHS, 32), lambda n: (0, 0)),
        out_shape=jax.ShapeDtypeStruct((N_GRAPHS, 32), jnp.float32),
    )(agg3, i3, gcn_b)


def _make_m(wk, bk, f, c):
    wk_aug = jnp.concatenate(
        [wk, bk[None, :], jnp.zeros((1, f * c), jnp.float32)], axis=0)
    a = wk_aug.reshape(32, f, c)
    if f < 32:
        a = jnp.pad(a, ((0, 0), (0, 32 - f), (0, 0)))
    return a.reshape(32 * 32, c)


def _pad2(w, rows, cols):
    return jnp.pad(w, ((0, rows - w.shape[0]), (0, cols - w.shape[1])))


def kernel(x, edge_index, e, i, m1_W1, m1_b1, m1_W2, m1_b2, m1_Wk, m1_bk,
           m1_root, m1_bias, m2_W1, m2_b1, m2_W2, m2_b2, m2_Wk, m2_bk,
           m2_root, m2_bias, gcn_W, gcn_b):
    src = edge_index[0]
    dst = edge_index[1]
    src2 = jnp.concatenate(
        [src, jnp.zeros((EPAD - E,), jnp.int32)]).reshape(NW, PERW)
    dpad = jnp.concatenate([dst, jnp.full((EPAD - E,), NPAD - 1, jnp.int32)])
    d0 = jnp.where(dpad < HALF, dpad, HALF)
    d1r = dpad - HALF
    d1 = jnp.where((d1r >= 0) & (d1r < HALF), d1r, HALF)
    dsts = jnp.stack([d0, d1]).reshape(NC, NW, CH, CHUNK)
    z32 = jnp.zeros((IMG, 32), jnp.bfloat16)

    e_pad = jnp.pad(e, ((0, EPAD - E), (0, 0)))
    x32 = jnp.pad(x, ((0, 0), (0, 32 - F_IN))).astype(jnp.bfloat16)

    m1w1 = _pad2(m1_W1, D_E, 32)
    m1b1 = jnp.pad(m1_b1, (0, 2)).reshape(1, 32)
    m1w2 = _pad2(m1_W2, 32, 32)
    m1b2 = jnp.pad(m1_b2, (0, 2)).reshape(1, 32)
    m1m = _make_m(m1_Wk, m1_bk, F_IN, 32).astype(jnp.bfloat16)
    m2w1 = _pad2(m2_W1, D_E, 32)
    m2b1 = jnp.pad(m2_b1, (0, 2)).reshape(1, 32)
    m2w2 = _pad2(m2_W2, 32, 32)
    m2b2 = jnp.pad(m2_b2, (0, 2)).reshape(1, 32)
    m2m = _make_m(m2_Wk, m2_bk, 32, 64).astype(jnp.bfloat16)

    eye = jnp.eye(32, dtype=jnp.bfloat16)
    rx = jnp.kron(jnp.ones((1, 32), jnp.bfloat16), eye)
    rh = jnp.kron(eye, jnp.ones((1, 32), jnp.bfloat16))

    xg = _sc_gather(x32, src2)
    (msg1,) = _edge_call(e_pad, xg, m1w1, m1b1, m1w2, m1b2, rx, rh, m1m, 32)
    agg1 = _sc_scatter(msg1, dsts, z32, 32)
    x_n = jnp.pad(x, ((0, NPAD - N), (0, 0)))
    h1 = _node1_call(agg1, x_n, m1_root, m1_bias.reshape(1, 32))

    h1g = _sc_gather(h1, src2)
    msg2a, msg2b = _edge_call(e_pad, h1g, m2w1, m2b1, m2w2, m2b2, rx, rh, m2m, 64)
    agg2a = _sc_scatter(msg2a, dsts, z32, 32)
    agg2b = _sc_scatter(msg2b, dsts, z32, 32)
    hw = _node2_call(agg2a, agg2b, h1, m2_root, m2_bias.reshape(1, 64), gcn_W)

    agg3 = _sc_gather_scatter(hw, src2, dsts, z32)
    i_pad = jnp.concatenate(
        [i, jnp.full((NPAD - N,), N_GRAPHS, jnp.int32)]).reshape(NPAD // BN, 1, BN)
    return _pool_call(agg3, i_pad, gcn_b.reshape(1, 32))


# speedup vs baseline: 3.1254x; 1.0097x over previous
import functools

import jax
import jax.numpy as jnp
from jax import lax
from jax.experimental import pallas as pl
from jax.experimental.pallas import tpu as pltpu
from jax.experimental.pallas import tpu_sc as plsc

N = 20000
E = 50000
F_IN = 30
D_E = 16
N_GRAPHS = 512

T_E = 512
EPAD = 53248
BN = 2000
NPAD = 20480

NC = 2
NS = 16
NW = NC * NS
CHUNK = 128
CH = EPAD // (NW * CHUNK)
PERW = CH * CHUNK
HALF = NPAD // 2
IMG = HALF + 128
SSTRIPE = IMG // NS
DSTRIPE = HALF // NS


def _relu(v):
    return jnp.maximum(v, 0.0)


def _sc_mesh():
    return plsc.VectorSubcoreMesh(core_axis_name="c", subcore_axis_name="s")


def _sc_gather(table, idx2):
    @functools.partial(
        pl.kernel,
        out_type=jax.ShapeDtypeStruct((EPAD, 32), jnp.bfloat16),
        mesh=_sc_mesh(),
        compiler_params=pltpu.CompilerParams(use_tc_tiling_on_sc=False),
        scratch_types=[
            pltpu.VMEM((PERW,), jnp.int32),
            pltpu.VMEM((PERW, 32), jnp.bfloat16),
            pltpu.SemaphoreType.DMA,
        ],
    )
    def k(table_hbm, idx_hbm, out_hbm, idx_v, rows_v, sem):
        wid = lax.axis_index("s") * NC + lax.axis_index("c")
        pltpu.sync_copy(idx_hbm.at[wid], idx_v)
        pltpu.async_copy(table_hbm.at[idx_v], rows_v, sem).wait()
        pltpu.sync_copy(rows_v, out_hbm.at[pl.ds(wid * PERW, PERW)])

    return k(table, idx2)


def _sc_scatter(msg, dsts, zrows, c):
    @functools.partial(
        pl.kernel,
        out_type=jax.ShapeDtypeStruct((NPAD, c), jnp.bfloat16),
        mesh=_sc_mesh(),
        compiler_params=pltpu.CompilerParams(use_tc_tiling_on_sc=False),
        scratch_types=[
            pltpu.VMEM((CH, CHUNK), jnp.int32),
            pltpu.VMEM((PERW, c), jnp.bfloat16),
            pltpu.VMEM_SHARED((IMG, c), jnp.bfloat16),
        ],
    )
    def k(msg_hbm, dst_hbm, z_hbm, out_hbm, dst_v, msg_v, shared):
        cc = lax.axis_index("c")
        s = lax.axis_index("s")
        pltpu.sync_copy(z_hbm.at[pl.ds(s * SSTRIPE, SSTRIPE)],
                        shared.at[pl.ds(s * SSTRIPE, SSTRIPE)])
        plsc.subcore_barrier()
        for half in range(2):
            w = s * 2 + half
            pltpu.sync_copy(dst_hbm.at[cc].at[w], dst_v)
            pltpu.sync_copy(msg_hbm.at[pl.ds(w * PERW, PERW)], msg_v)
            for j in range(CH):
                pltpu.sync_copy(msg_v.at[pl.ds(j * CHUNK, CHUNK)],
                                shared.at[dst_v.at[j]], add=True)
        plsc.subcore_barrier()
        pltpu.sync_copy(shared.at[pl.ds(s * DSTRIPE, DSTRIPE)],
                        out_hbm.at[pl.ds(cc * HALF + s * DSTRIPE, DSTRIPE)])

    return k(msg, dsts, zrows)


def _sc_gather_scatter(hw, src2, dsts, zrows):
    @functools.partial(
        pl.kernel,
        out_type=jax.ShapeDtypeStruct((NPAD, 32), jnp.bfloat16),
        mesh=_sc_mesh(),
        compiler_params=pltpu.CompilerParams(use_tc_tiling_on_sc=False),
        scratch_types=[
            pltpu.VMEM((PERW,), jnp.int32),
            pltpu.VMEM((CH, CHUNK), jnp.int32),
            pltpu.VMEM((PERW, 32), jnp.bfloat16),
            pltpu.VMEM_SHARED((IMG, 32), jnp.bfloat16),
            pltpu.SemaphoreType.DMA,
        ],
    )
    def k(hw_hbm, src_hbm, dst_hbm, z_hbm, out_hbm, src_v, dst_v, rows_v,
          shared, sem):
        cc = lax.axis_index("c")
        s = lax.axis_index("s")
        pltpu.sync_copy(z_hbm.at[pl.ds(s * SSTRIPE, SSTRIPE)],
                        shared.at[pl.ds(s * SSTRIPE, SSTRIPE)])
        plsc.subcore_barrier()
        for half in range(2):
            w = s * 2 + half
            pltpu.sync_copy(src_hbm.at[w], src_v)
            pltpu.sync_copy(dst_hbm.at[cc].at[w], dst_v)
            pltpu.async_copy(hw_hbm.at[src_v], rows_v, sem).wait()
            for j in range(CH):
                pltpu.sync_copy(rows_v.at[pl.ds(j * CHUNK, CHUNK)],
                                shared.at[dst_v.at[j]], add=True)
        plsc.subcore_barrier()
        pltpu.sync_copy(shared.at[pl.ds(s * DSTRIPE, DSTRIPE)],
                        out_hbm.at[pl.ds(cc * HALF + s * DSTRIPE, DSTRIPE)])

    return k(hw, src2, dsts, zrows)


def _edge_body(e_ref, xg_ref, w1_ref, b1_ref, w2_ref, b2_ref, rx_ref, rh_ref,
               m_ref, *o_refs, c):
    h = _relu(jnp.dot(e_ref[...], w1_ref[...],
                      preferred_element_type=jnp.float32) + b1_ref[...])
    h = _relu(jnp.dot(h, w2_ref[...],
                      preferred_element_type=jnp.float32) + b2_ref[...])
    lane = jax.lax.broadcasted_iota(jnp.int32, h.shape, 1)
    h = jnp.where(lane == 30, 1.0, h)
    xt = jnp.dot(xg_ref[...], rx_ref[...],
                 preferred_element_type=jnp.float32)
    hr = jnp.dot(h.astype(jnp.bfloat16), rh_ref[...],
                 preferred_element_type=jnp.float32)
    p = (xt * hr).astype(jnp.bfloat16)
    msg = jnp.dot(p, m_ref[...], preferred_element_type=jnp.float32)
    row = jax.lax.broadcasted_iota(jnp.int32, msg.shape, 0) + pl.program_id(0) * T_E
    msg = jnp.where(row < E, msg, 0.0)
    for g, o_ref in enumerate(o_refs):
        o_ref[...] = msg[:, g * 32:(g + 1) * 32].astype(jnp.bfloat16)


def _edge_call(e_pad, xg, w1p, b1p, w2p, b2p, rx, rh, mp, c):
    grid = EPAD // T_E
    return pl.pallas_call(
        functools.partial(_edge_body, c=c),
        grid=(grid,),
        in_specs=[
            pl.BlockSpec((T_E, D_E), lambda n: (n, 0)),
            pl.BlockSpec((T_E, 32), lambda n: (n, 0)),
            pl.BlockSpec((D_E, 32), lambda n: (0, 0)),
            pl.BlockSpec((1, 32), lambda n: (0, 0)),
            pl.BlockSpec((32, 32), lambda n: (0, 0)),
            pl.BlockSpec((1, 32), lambda n: (0, 0)),
            pl.BlockSpec((32, 32 * 32), lambda n: (0, 0)),
            pl.BlockSpec((32, 32 * 32), lambda n: (0, 0)),
            pl.BlockSpec((32 * 32, c), lambda n: (0, 0)),
        ],
        out_specs=[pl.BlockSpec((T_E, 32), lambda n: (n, 0))] * (c // 32),
        out_shape=[jax.ShapeDtypeStruct((EPAD, 32), jnp.bfloat16)] * (c // 32),
    )(e_pad, xg, w1p, b1p, w2p, b2p, rx, rh, mp)


def _node1_body(agg_ref, x_ref, root_ref, bias_ref, o_ref):
    o_ref[...] = _relu(
        agg_ref[...].astype(jnp.float32)
        + jnp.dot(x_ref[...], root_ref[...], preferred_element_type=jnp.float32)
        + bias_ref[...]).astype(jnp.bfloat16)


def _node1_call(agg, x, root, bias):
    grid = N // BN
    return pl.pallas_call(
        _node1_body,
        grid=(grid,),
        in_specs=[
            pl.BlockSpec((BN, 32), lambda n: (n, 0)),
            pl.BlockSpec((BN, F_IN), lambda n: (n, 0)),
            pl.BlockSpec((F_IN, 32), lambda n: (0, 0)),
            pl.BlockSpec((1, 32), lambda n: (0, 0)),
        ],
        out_specs=pl.BlockSpec((BN, 32), lambda n: (n, 0)),
        out_shape=jax.ShapeDtypeStruct((N, 32), jnp.bfloat16),
    )(agg, x, root, bias)


def _node2_body(agga_ref, aggb_ref, h1_ref, root_ref, bias_ref, gw_ref, o_ref):
    rt = jnp.dot(h1_ref[...].astype(jnp.float32), root_ref[...],
                 preferred_element_type=jnp.float32)
    h2a = _relu(agga_ref[...].astype(jnp.float32) + rt[:, :32]
                + bias_ref[...][:, :32])
    h2b = _relu(aggb_ref[...].astype(jnp.float32) + rt[:, 32:]
                + bias_ref[...][:, 32:])
    o_ref[...] = (
        jnp.dot(h2a, gw_ref[...][:32], preferred_element_type=jnp.float32)
        + jnp.dot(h2b, gw_ref[...][32:], preferred_element_type=jnp.float32)
    ).astype(jnp.bfloat16)


def _node2_call(agga, aggb, h1, root, bias, gw):
    grid = N // BN
    return pl.pallas_call(
        _node2_body,
        grid=(grid,),
        in_specs=[
            pl.BlockSpec((BN, 32), lambda n: (n, 0)),
            pl.BlockSpec((BN, 32), lambda n: (n, 0)),
            pl.BlockSpec((BN, 32), lambda n: (n, 0)),
            pl.BlockSpec((32, 64), lambda n: (0, 0)),
            pl.BlockSpec((1, 64), lambda n: (0, 0)),
            pl.BlockSpec((64, 32), lambda n: (0, 0)),
        ],
        out_specs=pl.BlockSpec((BN, 32), lambda n: (n, 0)),
        out_shape=jax.ShapeDtypeStruct((N, 32), jnp.bfloat16),
    )(agga, aggb, h1, root, bias, gw)


def _pool_body(agg_ref, i_ref, b_ref, o_ref):
    n = pl.program_id(0)
    h3 = _relu(agg_ref[...].astype(jnp.float32) + b_ref[...])
    iv = i_ref[0]
    onehot = jnp.where(
        jax.lax.broadcasted_iota(jnp.int32, (N_GRAPHS, BN), 0) == iv, 1.0, 0.0)
    contrib = jnp.dot(onehot, h3, preferred_element_type=jnp.float32)

    @pl.when(n == 0)
    def _():
        o_ref[...] = jnp.zeros_like(o_ref)

    o_ref[...] += contrib


def _pool_call(agg3, i3, gcn_b):
    grid = N // BN
    return pl.pallas_call(
        _pool_body,
        grid=(grid,),
        in_specs=[
            pl.BlockSpec((BN, 32), lambda n: (n, 0)),
            pl.BlockSpec((1, 1, BN), lambda n: (n, 0, 0)),
            pl.BlockSpec((1, 32), lambda n: (0, 0)),
        ],
        out_specs=pl.BlockSpec((N_GRAPHS, 32), lambda n: (0, 0)),
        out_shape=jax.ShapeDtypeStruct((N_GRAPHS, 32), jnp.float32),
    )(agg3, i3, gcn_b)


def _make_m(wk, bk, f, c):
    wk_aug = jnp.concatenate(
        [wk, bk[None, :], jnp.zeros((1, f * c), jnp.float32)], axis=0)
    a = wk_aug.reshape(32, f, c)
    if f < 32:
        a = jnp.pad(a, ((0, 0), (0, 32 - f), (0, 0)))
    return a.reshape(32 * 32, c)


def _pad2(w, rows, cols):
    return jnp.pad(w, ((0, rows - w.shape[0]), (0, cols - w.shape[1])))


def kernel(x, edge_index, e, i, m1_W1, m1_b1, m1_W2, m1_b2, m1_Wk, m1_bk,
           m1_root, m1_bias, m2_W1, m2_b1, m2_W2, m2_b2, m2_Wk, m2_bk,
           m2_root, m2_bias, gcn_W, gcn_b):
    src = edge_index[0]
    dst = edge_index[1]
    src2 = jnp.concatenate(
        [src, jnp.zeros((EPAD - E,), jnp.int32)]).reshape(NW, PERW)
    dpad = jnp.concatenate([dst, jnp.full((EPAD - E,), NPAD - 1, jnp.int32)])
    d0 = jnp.where(dpad < HALF, dpad, HALF)
    d1r = dpad - HALF
    d1 = jnp.where((d1r >= 0) & (d1r < HALF), d1r, HALF)
    dsts = jnp.stack([d0, d1]).reshape(NC, NW, CH, CHUNK)
    z32 = jnp.zeros((IMG, 32), jnp.bfloat16)

    e_pad = jnp.pad(e, ((0, EPAD - E), (0, 0))).astype(jnp.bfloat16)
    x32 = jnp.pad(x, ((0, 0), (0, 32 - F_IN))).astype(jnp.bfloat16)

    m1w1 = _pad2(m1_W1, D_E, 32).astype(jnp.bfloat16)
    m1b1 = jnp.pad(m1_b1, (0, 2)).reshape(1, 32)
    m1w2 = _pad2(m1_W2, 32, 32)
    m1b2 = jnp.pad(m1_b2, (0, 2)).reshape(1, 32)
    m1m = _make_m(m1_Wk, m1_bk, F_IN, 32).astype(jnp.bfloat16)
    m2w1 = _pad2(m2_W1, D_E, 32).astype(jnp.bfloat16)
    m2b1 = jnp.pad(m2_b1, (0, 2)).reshape(1, 32)
    m2w2 = _pad2(m2_W2, 32, 32)
    m2b2 = jnp.pad(m2_b2, (0, 2)).reshape(1, 32)
    m2m = _make_m(m2_Wk, m2_bk, 32, 64).astype(jnp.bfloat16)

    eye = jnp.eye(32, dtype=jnp.bfloat16)
    rx = jnp.kron(jnp.ones((1, 32), jnp.bfloat16), eye)
    rh = jnp.kron(eye, jnp.ones((1, 32), jnp.bfloat16))

    xg = _sc_gather(x32, src2)
    (msg1,) = _edge_call(e_pad, xg, m1w1, m1b1, m1w2, m1b2, rx, rh, m1m, 32)
    agg1 = _sc_scatter(msg1, dsts, z32, 32)
    h1 = _node1_call(agg1, x, m1_root, m1_bias.reshape(1, 32))

    h1g = _sc_gather(h1, src2)
    msg2a, msg2b = _edge_call(e_pad, h1g, m2w1, m2b1, m2w2, m2b2, rx, rh, m2m, 64)
    agg2a = _sc_scatter(msg2a, dsts, z32, 32)
    agg2b = _sc_scatter(msg2b, dsts, z32, 32)
    hw = _node2_call(agg2a, agg2b, h1, m2_root, m2_bias.reshape(1, 64), gcn_W)

    agg3 = _sc_gather_scatter(hw, src2, dsts, z32)
    i3 = i.reshape(N // BN, 1, BN)
    return _pool_call(agg3, i3, gcn_b.reshape(1, 32))


# speedup vs baseline: 3.3868x; 1.0836x over previous
import functools

import jax
import jax.numpy as jnp
from jax import lax
from jax.experimental import pallas as pl
from jax.experimental.pallas import tpu as pltpu
from jax.experimental.pallas import tpu_sc as plsc

N = 20000
E = 50000
F_IN = 30
D_E = 16
N_GRAPHS = 512

T_E = 1024
EPAD = 53248
BN = 2000
NPAD = 20480

NC = 2
NS = 16
NW = NC * NS
CHUNK = 128
CH = EPAD // (NW * CHUNK)
PERW = CH * CHUNK
HALF = NPAD // 2
IMG = HALF + 128
SSTRIPE = IMG // NS
DSTRIPE = HALF // NS


def _relu(v):
    return jnp.maximum(v, 0.0)


def _sc_mesh():
    return plsc.VectorSubcoreMesh(core_axis_name="c", subcore_axis_name="s")


def _sc_gather(table, idx2):
    @functools.partial(
        pl.kernel,
        out_type=jax.ShapeDtypeStruct((EPAD, 32), jnp.bfloat16),
        mesh=_sc_mesh(),
        compiler_params=pltpu.CompilerParams(use_tc_tiling_on_sc=False),
        scratch_types=[
            pltpu.VMEM((PERW,), jnp.int32),
            pltpu.VMEM((PERW, 32), jnp.bfloat16),
            pltpu.SemaphoreType.DMA,
        ],
    )
    def k(table_hbm, idx_hbm, out_hbm, idx_v, rows_v, sem):
        wid = lax.axis_index("s") * NC + lax.axis_index("c")
        pltpu.sync_copy(idx_hbm.at[wid], idx_v)
        pltpu.async_copy(table_hbm.at[idx_v], rows_v, sem).wait()
        pltpu.sync_copy(rows_v, out_hbm.at[pl.ds(wid * PERW, PERW)])

    return k(table, idx2)


def _sc_scatter(msg, dsts, zrows, c):
    @functools.partial(
        pl.kernel,
        out_type=jax.ShapeDtypeStruct((NPAD, c), jnp.bfloat16),
        mesh=_sc_mesh(),
        compiler_params=pltpu.CompilerParams(use_tc_tiling_on_sc=False),
        scratch_types=[
            pltpu.VMEM((CH, CHUNK), jnp.int32),
            pltpu.VMEM((PERW, c), jnp.bfloat16),
            pltpu.VMEM_SHARED((IMG, c), jnp.bfloat16),
        ],
    )
    def k(msg_hbm, dst_hbm, z_hbm, out_hbm, dst_v, msg_v, shared):
        cc = lax.axis_index("c")
        s = lax.axis_index("s")
        pltpu.sync_copy(z_hbm.at[pl.ds(s * SSTRIPE, SSTRIPE)],
                        shared.at[pl.ds(s * SSTRIPE, SSTRIPE)])
        plsc.subcore_barrier()
        for half in range(2):
            w = s * 2 + half
            pltpu.sync_copy(dst_hbm.at[cc].at[w], dst_v)
            pltpu.sync_copy(msg_hbm.at[pl.ds(w * PERW, PERW)], msg_v)
            for j in range(CH):
                pltpu.sync_copy(msg_v.at[pl.ds(j * CHUNK, CHUNK)],
                                shared.at[dst_v.at[j]], add=True)
        plsc.subcore_barrier()
        pltpu.sync_copy(shared.at[pl.ds(s * DSTRIPE, DSTRIPE)],
                        out_hbm.at[pl.ds(cc * HALF + s * DSTRIPE, DSTRIPE)])

    return k(msg, dsts, zrows)


def _sc_gather_scatter(hw, src2, dsts, zrows):
    @functools.partial(
        pl.kernel,
        out_type=jax.ShapeDtypeStruct((NPAD, 32), jnp.bfloat16),
        mesh=_sc_mesh(),
        compiler_params=pltpu.CompilerParams(use_tc_tiling_on_sc=False),
        scratch_types=[
            pltpu.VMEM((PERW,), jnp.int32),
            pltpu.VMEM((CH, CHUNK), jnp.int32),
            pltpu.VMEM((PERW, 32), jnp.bfloat16),
            pltpu.VMEM_SHARED((IMG, 32), jnp.bfloat16),
            pltpu.SemaphoreType.DMA,
        ],
    )
    def k(hw_hbm, src_hbm, dst_hbm, z_hbm, out_hbm, src_v, dst_v, rows_v,
          shared, sem):
        cc = lax.axis_index("c")
        s = lax.axis_index("s")
        pltpu.sync_copy(z_hbm.at[pl.ds(s * SSTRIPE, SSTRIPE)],
                        shared.at[pl.ds(s * SSTRIPE, SSTRIPE)])
        plsc.subcore_barrier()
        for half in range(2):
            w = s * 2 + half
            pltpu.sync_copy(src_hbm.at[w], src_v)
            pltpu.sync_copy(dst_hbm.at[cc].at[w], dst_v)
            pltpu.async_copy(hw_hbm.at[src_v], rows_v, sem).wait()
            for j in range(CH):
                pltpu.sync_copy(rows_v.at[pl.ds(j * CHUNK, CHUNK)],
                                shared.at[dst_v.at[j]], add=True)
        plsc.subcore_barrier()
        pltpu.sync_copy(shared.at[pl.ds(s * DSTRIPE, DSTRIPE)],
                        out_hbm.at[pl.ds(cc * HALF + s * DSTRIPE, DSTRIPE)])

    return k(hw, src2, dsts, zrows)


def _edge_body(e_ref, xg_ref, w1_ref, b1_ref, w2_ref, b2_ref, rx_ref, rh_ref,
               m_ref, *o_refs, c):
    h = _relu(jnp.dot(e_ref[...], w1_ref[...],
                      preferred_element_type=jnp.float32) + b1_ref[...])
    h = _relu(jnp.dot(h, w2_ref[...],
                      preferred_element_type=jnp.float32) + b2_ref[...])
    lane = jax.lax.broadcasted_iota(jnp.int32, h.shape, 1)
    h = jnp.where(lane == 30, 1.0, h)
    xt = jnp.dot(xg_ref[...], rx_ref[...],
                 preferred_element_type=jnp.float32)
    hr = jnp.dot(h.astype(jnp.bfloat16), rh_ref[...],
                 preferred_element_type=jnp.float32)
    p = (xt * hr).astype(jnp.bfloat16)
    msg = jnp.dot(p, m_ref[...], preferred_element_type=jnp.float32)
    row = jax.lax.broadcasted_iota(jnp.int32, msg.shape, 0) + pl.program_id(0) * T_E
    msg = jnp.where(row < E, msg, 0.0)
    for g, o_ref in enumerate(o_refs):
        o_ref[...] = msg[:, g * 32:(g + 1) * 32].astype(jnp.bfloat16)


def _edge_call(e_pad, xg, w1p, b1p, w2p, b2p, rx, rh, mp, c):
    grid = EPAD // T_E
    return pl.pallas_call(
        functools.partial(_edge_body, c=c),
        grid=(grid,),
        in_specs=[
            pl.BlockSpec((T_E, D_E), lambda n: (n, 0)),
            pl.BlockSpec((T_E, 32), lambda n: (n, 0)),
            pl.BlockSpec((D_E, 32), lambda n: (0, 0)),
            pl.BlockSpec((1, 32), lambda n: (0, 0)),
            pl.BlockSpec((32, 32), lambda n: (0, 0)),
            pl.BlockSpec((1, 32), lambda n: (0, 0)),
            pl.BlockSpec((32, 32 * 32), lambda n: (0, 0)),
            pl.BlockSpec((32, 32 * 32), lambda n: (0, 0)),
            pl.BlockSpec((32 * 32, c), lambda n: (0, 0)),
        ],
        out_specs=[pl.BlockSpec((T_E, 32), lambda n: (n, 0))] * (c // 32),
        out_shape=[jax.ShapeDtypeStruct((EPAD, 32), jnp.bfloat16)] * (c // 32),
    )(e_pad, xg, w1p, b1p, w2p, b2p, rx, rh, mp)


def _node1_body(agg_ref, x_ref, root_ref, bias_ref, o_ref):
    o_ref[...] = _relu(
        agg_ref[...].astype(jnp.float32)
        + jnp.dot(x_ref[...], root_ref[...], preferred_element_type=jnp.float32)
        + bias_ref[...]).astype(jnp.bfloat16)


def _node1_call(agg, x, root, bias):
    grid = N // BN
    return pl.pallas_call(
        _node1_body,
        grid=(grid,),
        in_specs=[
            pl.BlockSpec((BN, 32), lambda n: (n, 0)),
            pl.BlockSpec((BN, F_IN), lambda n: (n, 0)),
            pl.BlockSpec((F_IN, 32), lambda n: (0, 0)),
            pl.BlockSpec((1, 32), lambda n: (0, 0)),
        ],
        out_specs=pl.BlockSpec((BN, 32), lambda n: (n, 0)),
        out_shape=jax.ShapeDtypeStruct((N, 32), jnp.bfloat16),
    )(agg, x, root, bias)


def _node2_body(agga_ref, aggb_ref, h1_ref, root_ref, bias_ref, gw_ref, o_ref):
    rt = jnp.dot(h1_ref[...].astype(jnp.float32), root_ref[...],
                 preferred_element_type=jnp.float32)
    h2a = _relu(agga_ref[...].astype(jnp.float32) + rt[:, :32]
                + bias_ref[...][:, :32])
    h2b = _relu(aggb_ref[...].astype(jnp.float32) + rt[:, 32:]
                + bias_ref[...][:, 32:])
    o_ref[...] = (
        jnp.dot(h2a, gw_ref[...][:32], preferred_element_type=jnp.float32)
        + jnp.dot(h2b, gw_ref[...][32:], preferred_element_type=jnp.float32)
    ).astype(jnp.bfloat16)


def _node2_call(agga, aggb, h1, root, bias, gw):
    grid = N // BN
    return pl.pallas_call(
        _node2_body,
        grid=(grid,),
        in_specs=[
            pl.BlockSpec((BN, 32), lambda n: (n, 0)),
            pl.BlockSpec((BN, 32), lambda n: (n, 0)),
            pl.BlockSpec((BN, 32), lambda n: (n, 0)),
            pl.BlockSpec((32, 64), lambda n: (0, 0)),
            pl.BlockSpec((1, 64), lambda n: (0, 0)),
            pl.BlockSpec((64, 32), lambda n: (0, 0)),
        ],
        out_specs=pl.BlockSpec((BN, 32), lambda n: (n, 0)),
        out_shape=jax.ShapeDtypeStruct((N, 32), jnp.bfloat16),
    )(agga, aggb, h1, root, bias, gw)


def _pool_body(agg_ref, i_ref, b_ref, o_ref):
    n = pl.program_id(0)
    h3 = _relu(agg_ref[...].astype(jnp.float32) + b_ref[...])
    iv = i_ref[0]
    onehot = jnp.where(
        jax.lax.broadcasted_iota(jnp.int32, (N_GRAPHS, BN), 0) == iv, 1.0, 0.0)
    contrib = jnp.dot(onehot, h3, preferred_element_type=jnp.float32)

    @pl.when(n == 0)
    def _():
        o_ref[...] = jnp.zeros_like(o_ref)

    o_ref[...] += contrib


def _pool_call(agg3, i3, gcn_b):
    grid = N // BN
    return pl.pallas_call(
        _pool_body,
        grid=(grid,),
        in_specs=[
            pl.BlockSpec((BN, 32), lambda n: (n, 0)),
            pl.BlockSpec((1, 1, BN), lambda n: (n, 0, 0)),
            pl.BlockSpec((1, 32), lambda n: (0, 0)),
        ],
        out_specs=pl.BlockSpec((N_GRAPHS, 32), lambda n: (0, 0)),
        out_shape=jax.ShapeDtypeStruct((N_GRAPHS, 32), jnp.float32),
    )(agg3, i3, gcn_b)


def _make_m(wk, bk, f, c):
    wk_aug = jnp.concatenate(
        [wk, bk[None, :], jnp.zeros((1, f * c), jnp.float32)], axis=0)
    a = wk_aug.reshape(32, f, c)
    if f < 32:
        a = jnp.pad(a, ((0, 0), (0, 32 - f), (0, 0)))
    return a.reshape(32 * 32, c)


def _pad2(w, rows, cols):
    return jnp.pad(w, ((0, rows - w.shape[0]), (0, cols - w.shape[1])))


def kernel(x, edge_index, e, i, m1_W1, m1_b1, m1_W2, m1_b2, m1_Wk, m1_bk,
           m1_root, m1_bias, m2_W1, m2_b1, m2_W2, m2_b2, m2_Wk, m2_bk,
           m2_root, m2_bias, gcn_W, gcn_b):
    src = edge_index[0]
    dst = edge_index[1]
    src2 = jnp.concatenate(
        [src, jnp.zeros((EPAD - E,), jnp.int32)]).reshape(NW, PERW)
    dpad = jnp.concatenate([dst, jnp.full((EPAD - E,), NPAD - 1, jnp.int32)])
    d0 = jnp.where(dpad < HALF, dpad, HALF)
    d1r = dpad - HALF
    d1 = jnp.where((d1r >= 0) & (d1r < HALF), d1r, HALF)
    dsts = jnp.stack([d0, d1]).reshape(NC, NW, CH, CHUNK)
    z32 = jnp.zeros((IMG, 32), jnp.bfloat16)

    e_pad = jnp.pad(e, ((0, EPAD - E), (0, 0))).astype(jnp.bfloat16)
    x32 = jnp.pad(x, ((0, 0), (0, 32 - F_IN))).astype(jnp.bfloat16)

    m1w1 = _pad2(m1_W1, D_E, 32).astype(jnp.bfloat16)
    m1b1 = jnp.pad(m1_b1, (0, 2)).reshape(1, 32)
    m1w2 = _pad2(m1_W2, 32, 32)
    m1b2 = jnp.pad(m1_b2, (0, 2)).reshape(1, 32)
    m1m = _make_m(m1_Wk, m1_bk, F_IN, 32).astype(jnp.bfloat16)
    m2w1 = _pad2(m2_W1, D_E, 32).astype(jnp.bfloat16)
    m2b1 = jnp.pad(m2_b1, (0, 2)).reshape(1, 32)
    m2w2 = _pad2(m2_W2, 32, 32)
    m2b2 = jnp.pad(m2_b2, (0, 2)).reshape(1, 32)
    m2m = _make_m(m2_Wk, m2_bk, 32, 64).astype(jnp.bfloat16)

    eye = jnp.eye(32, dtype=jnp.bfloat16)
    rx = jnp.kron(jnp.ones((1, 32), jnp.bfloat16), eye)
    rh = jnp.kron(eye, jnp.ones((1, 32), jnp.bfloat16))

    xg = _sc_gather(x32, src2)
    (msg1,) = _edge_call(e_pad, xg, m1w1, m1b1, m1w2, m1b2, rx, rh, m1m, 32)
    agg1 = _sc_scatter(msg1, dsts, z32, 32)
    h1 = _node1_call(agg1, x, m1_root, m1_bias.reshape(1, 32))

    h1g = _sc_gather(h1, src2)
    msg2a, msg2b = _edge_call(e_pad, h1g, m2w1, m2b1, m2w2, m2b2, rx, rh, m2m, 64)
    agg2a = _sc_scatter(msg2a, dsts, z32, 32)
    agg2b = _sc_scatter(msg2b, dsts, z32, 32)
    hw = _node2_call(agg2a, agg2b, h1, m2_root, m2_bias.reshape(1, 64), gcn_W)

    agg3 = _sc_gather_scatter(hw, src2, dsts, z32)
    i3 = i.reshape(N // BN, 1, BN)
    return _pool_call(agg3, i3, gcn_b.reshape(1, 32))


# speedup vs baseline: 3.5011x; 1.0337x over previous
import functools

import jax
import jax.numpy as jnp
from jax import lax
from jax.experimental import pallas as pl
from jax.experimental.pallas import tpu as pltpu
from jax.experimental.pallas import tpu_sc as plsc

N = 20000
E = 50000
F_IN = 30
D_E = 16
N_GRAPHS = 512

T_E = 2048
EPAD = 53248
BN = 2000
NPAD = 20480

NC = 2
NS = 16
NW = NC * NS
CHUNK = 128
CH = EPAD // (NW * CHUNK)
PERW = CH * CHUNK
HALF = NPAD // 2
IMG = HALF + 128
SSTRIPE = IMG // NS
DSTRIPE = HALF // NS


def _relu(v):
    return jnp.maximum(v, 0.0)


def _sc_mesh():
    return plsc.VectorSubcoreMesh(core_axis_name="c", subcore_axis_name="s")


def _sc_gather(table, idx2):
    @functools.partial(
        pl.kernel,
        out_type=jax.ShapeDtypeStruct((EPAD, 32), jnp.bfloat16),
        mesh=_sc_mesh(),
        compiler_params=pltpu.CompilerParams(use_tc_tiling_on_sc=False),
        scratch_types=[
            pltpu.VMEM((PERW,), jnp.int32),
            pltpu.VMEM((PERW, 32), jnp.bfloat16),
            pltpu.SemaphoreType.DMA,
        ],
    )
    def k(table_hbm, idx_hbm, out_hbm, idx_v, rows_v, sem):
        wid = lax.axis_index("s") * NC + lax.axis_index("c")
        pltpu.sync_copy(idx_hbm.at[wid], idx_v)
        pltpu.async_copy(table_hbm.at[idx_v], rows_v, sem).wait()
        pltpu.sync_copy(rows_v, out_hbm.at[pl.ds(wid * PERW, PERW)])

    return k(table, idx2)


def _sc_scatter(msg, dsts, zrows, c):
    @functools.partial(
        pl.kernel,
        out_type=jax.ShapeDtypeStruct((NPAD, c), jnp.bfloat16),
        mesh=_sc_mesh(),
        compiler_params=pltpu.CompilerParams(use_tc_tiling_on_sc=False),
        scratch_types=[
            pltpu.VMEM((CH, CHUNK), jnp.int32),
            pltpu.VMEM((PERW, c), jnp.bfloat16),
            pltpu.VMEM_SHARED((IMG, c), jnp.bfloat16),
        ],
    )
    def k(msg_hbm, dst_hbm, z_hbm, out_hbm, dst_v, msg_v, shared):
        cc = lax.axis_index("c")
        s = lax.axis_index("s")
        pltpu.sync_copy(z_hbm.at[pl.ds(s * SSTRIPE, SSTRIPE)],
                        shared.at[pl.ds(s * SSTRIPE, SSTRIPE)])
        plsc.subcore_barrier()
        for half in range(2):
            w = s * 2 + half
            pltpu.sync_copy(dst_hbm.at[cc].at[w], dst_v)
            pltpu.sync_copy(msg_hbm.at[pl.ds(w * PERW, PERW)], msg_v)
            for j in range(CH):
                pltpu.sync_copy(msg_v.at[pl.ds(j * CHUNK, CHUNK)],
                                shared.at[dst_v.at[j]], add=True)
        plsc.subcore_barrier()
        pltpu.sync_copy(shared.at[pl.ds(s * DSTRIPE, DSTRIPE)],
                        out_hbm.at[pl.ds(cc * HALF + s * DSTRIPE, DSTRIPE)])

    return k(msg, dsts, zrows)


def _sc_gather_scatter(hw, src2, dsts, zrows):
    @functools.partial(
        pl.kernel,
        out_type=jax.ShapeDtypeStruct((NPAD, 32), jnp.bfloat16),
        mesh=_sc_mesh(),
        compiler_params=pltpu.CompilerParams(use_tc_tiling_on_sc=False),
        scratch_types=[
            pltpu.VMEM((PERW,), jnp.int32),
            pltpu.VMEM((CH, CHUNK), jnp.int32),
            pltpu.VMEM((PERW, 32), jnp.bfloat16),
            pltpu.VMEM_SHARED((IMG, 32), jnp.bfloat16),
            pltpu.SemaphoreType.DMA,
        ],
    )
    def k(hw_hbm, src_hbm, dst_hbm, z_hbm, out_hbm, src_v, dst_v, rows_v,
          shared, sem):
        cc = lax.axis_index("c")
        s = lax.axis_index("s")
        pltpu.sync_copy(z_hbm.at[pl.ds(s * SSTRIPE, SSTRIPE)],
                        shared.at[pl.ds(s * SSTRIPE, SSTRIPE)])
        plsc.subcore_barrier()
        for half in range(2):
            w = s * 2 + half
            pltpu.sync_copy(src_hbm.at[w], src_v)
            pltpu.sync_copy(dst_hbm.at[cc].at[w], dst_v)
            pltpu.async_copy(hw_hbm.at[src_v], rows_v, sem).wait()
            for j in range(CH):
                pltpu.sync_copy(rows_v.at[pl.ds(j * CHUNK, CHUNK)],
                                shared.at[dst_v.at[j]], add=True)
        plsc.subcore_barrier()
        pltpu.sync_copy(shared.at[pl.ds(s * DSTRIPE, DSTRIPE)],
                        out_hbm.at[pl.ds(cc * HALF + s * DSTRIPE, DSTRIPE)])

    return k(hw, src2, dsts, zrows)


def _edge_body(e_ref, xg_ref, w1_ref, b1_ref, w2_ref, b2_ref, rx_ref, rh_ref,
               m_ref, *o_refs, c):
    h = _relu(jnp.dot(e_ref[...], w1_ref[...],
                      preferred_element_type=jnp.float32) + b1_ref[...])
    h = _relu(jnp.dot(h, w2_ref[...],
                      preferred_element_type=jnp.float32) + b2_ref[...])
    lane = jax.lax.broadcasted_iota(jnp.int32, h.shape, 1)
    h = jnp.where(lane == 30, 1.0, h)
    xt = jnp.dot(xg_ref[...], rx_ref[...],
                 preferred_element_type=jnp.float32)
    hr = jnp.dot(h.astype(jnp.bfloat16), rh_ref[...],
                 preferred_element_type=jnp.float32)
    p = (xt * hr).astype(jnp.bfloat16)
    msg = jnp.dot(p, m_ref[...], preferred_element_type=jnp.float32)
    row = jax.lax.broadcasted_iota(jnp.int32, msg.shape, 0) + pl.program_id(0) * T_E
    msg = jnp.where(row < E, msg, 0.0)
    for g, o_ref in enumerate(o_refs):
        o_ref[...] = msg[:, g * 32:(g + 1) * 32].astype(jnp.bfloat16)


def _edge_call(e_pad, xg, w1p, b1p, w2p, b2p, rx, rh, mp, c):
    grid = EPAD // T_E
    return pl.pallas_call(
        functools.partial(_edge_body, c=c),
        grid=(grid,),
        in_specs=[
            pl.BlockSpec((T_E, D_E), lambda n: (n, 0)),
            pl.BlockSpec((T_E, 32), lambda n: (n, 0)),
            pl.BlockSpec((D_E, 32), lambda n: (0, 0)),
            pl.BlockSpec((1, 32), lambda n: (0, 0)),
            pl.BlockSpec((32, 32), lambda n: (0, 0)),
            pl.BlockSpec((1, 32), lambda n: (0, 0)),
            pl.BlockSpec((32, 32 * 32), lambda n: (0, 0)),
            pl.BlockSpec((32, 32 * 32), lambda n: (0, 0)),
            pl.BlockSpec((32 * 32, c), lambda n: (0, 0)),
        ],
        out_specs=[pl.BlockSpec((T_E, 32), lambda n: (n, 0))] * (c // 32),
        out_shape=[jax.ShapeDtypeStruct((EPAD, 32), jnp.bfloat16)] * (c // 32),
    )(e_pad, xg, w1p, b1p, w2p, b2p, rx, rh, mp)


def _node1_body(agg_ref, x_ref, root_ref, bias_ref, o_ref):
    o_ref[...] = _relu(
        agg_ref[...].astype(jnp.float32)
        + jnp.dot(x_ref[...], root_ref[...], preferred_element_type=jnp.float32)
        + bias_ref[...]).astype(jnp.bfloat16)


def _node1_call(agg, x, root, bias):
    grid = N // BN
    return pl.pallas_call(
        _node1_body,
        grid=(grid,),
        in_specs=[
            pl.BlockSpec((BN, 32), lambda n: (n, 0)),
            pl.BlockSpec((BN, F_IN), lambda n: (n, 0)),
            pl.BlockSpec((F_IN, 32), lambda n: (0, 0)),
            pl.BlockSpec((1, 32), lambda n: (0, 0)),
        ],
        out_specs=pl.BlockSpec((BN, 32), lambda n: (n, 0)),
        out_shape=jax.ShapeDtypeStruct((N, 32), jnp.bfloat16),
    )(agg, x, root, bias)


def _node2_body(agga_ref, aggb_ref, h1_ref, root_ref, bias_ref, gw_ref, o_ref):
    rt = jnp.dot(h1_ref[...].astype(jnp.float32), root_ref[...],
                 preferred_element_type=jnp.float32)
    h2a = _relu(agga_ref[...].astype(jnp.float32) + rt[:, :32]
                + bias_ref[...][:, :32])
    h2b = _relu(aggb_ref[...].astype(jnp.float32) + rt[:, 32:]
                + bias_ref[...][:, 32:])
    o_ref[...] = (
        jnp.dot(h2a, gw_ref[...][:32], preferred_element_type=jnp.float32)
        + jnp.dot(h2b, gw_ref[...][32:], preferred_element_type=jnp.float32)
    ).astype(jnp.bfloat16)


def _node2_call(agga, aggb, h1, root, bias, gw):
    grid = N // BN
    return pl.pallas_call(
        _node2_body,
        grid=(grid,),
        in_specs=[
            pl.BlockSpec((BN, 32), lambda n: (n, 0)),
            pl.BlockSpec((BN, 32), lambda n: (n, 0)),
            pl.BlockSpec((BN, 32), lambda n: (n, 0)),
            pl.BlockSpec((32, 64), lambda n: (0, 0)),
            pl.BlockSpec((1, 64), lambda n: (0, 0)),
            pl.BlockSpec((64, 32), lambda n: (0, 0)),
        ],
        out_specs=pl.BlockSpec((BN, 32), lambda n: (n, 0)),
        out_shape=jax.ShapeDtypeStruct((N, 32), jnp.bfloat16),
    )(agga, aggb, h1, root, bias, gw)


def _pool_body(agg_ref, i_ref, b_ref, o_ref):
    n = pl.program_id(0)
    h3 = _relu(agg_ref[...].astype(jnp.float32) + b_ref[...])
    iv = i_ref[0]
    onehot = jnp.where(
        jax.lax.broadcasted_iota(jnp.int32, (N_GRAPHS, BN), 0) == iv, 1.0, 0.0)
    contrib = jnp.dot(onehot, h3, preferred_element_type=jnp.float32)

    @pl.when(n == 0)
    def _():
        o_ref[...] = jnp.zeros_like(o_ref)

    o_ref[...] += contrib


def _pool_call(agg3, i3, gcn_b):
    grid = N // BN
    return pl.pallas_call(
        _pool_body,
        grid=(grid,),
        in_specs=[
            pl.BlockSpec((BN, 32), lambda n: (n, 0)),
            pl.BlockSpec((1, 1, BN), lambda n: (n, 0, 0)),
            pl.BlockSpec((1, 32), lambda n: (0, 0)),
        ],
        out_specs=pl.BlockSpec((N_GRAPHS, 32), lambda n: (0, 0)),
        out_shape=jax.ShapeDtypeStruct((N_GRAPHS, 32), jnp.float32),
    )(agg3, i3, gcn_b)


def _make_m(wk, bk, f, c):
    wk_aug = jnp.concatenate(
        [wk, bk[None, :], jnp.zeros((1, f * c), jnp.float32)], axis=0)
    a = wk_aug.reshape(32, f, c)
    if f < 32:
        a = jnp.pad(a, ((0, 0), (0, 32 - f), (0, 0)))
    return a.reshape(32 * 32, c)


def _pad2(w, rows, cols):
    return jnp.pad(w, ((0, rows - w.shape[0]), (0, cols - w.shape[1])))


def kernel(x, edge_index, e, i, m1_W1, m1_b1, m1_W2, m1_b2, m1_Wk, m1_bk,
           m1_root, m1_bias, m2_W1, m2_b1, m2_W2, m2_b2, m2_Wk, m2_bk,
           m2_root, m2_bias, gcn_W, gcn_b):
    src = edge_index[0]
    dst = edge_index[1]
    src2 = jnp.concatenate(
        [src, jnp.zeros((EPAD - E,), jnp.int32)]).reshape(NW, PERW)
    dpad = jnp.concatenate([dst, jnp.full((EPAD - E,), NPAD - 1, jnp.int32)])
    d0 = jnp.where(dpad < HALF, dpad, HALF)
    d1r = dpad - HALF
    d1 = jnp.where((d1r >= 0) & (d1r < HALF), d1r, HALF)
    dsts = jnp.stack([d0, d1]).reshape(NC, NW, CH, CHUNK)
    z32 = jnp.zeros((IMG, 32), jnp.bfloat16)

    e_pad = jnp.pad(e, ((0, EPAD - E), (0, 0))).astype(jnp.bfloat16)
    x32 = jnp.pad(x, ((0, 0), (0, 32 - F_IN))).astype(jnp.bfloat16)

    m1w1 = _pad2(m1_W1, D_E, 32).astype(jnp.bfloat16)
    m1b1 = jnp.pad(m1_b1, (0, 2)).reshape(1, 32)
    m1w2 = _pad2(m1_W2, 32, 32)
    m1b2 = jnp.pad(m1_b2, (0, 2)).reshape(1, 32)
    m1m = _make_m(m1_Wk, m1_bk, F_IN, 32).astype(jnp.bfloat16)
    m2w1 = _pad2(m2_W1, D_E, 32).astype(jnp.bfloat16)
    m2b1 = jnp.pad(m2_b1, (0, 2)).reshape(1, 32)
    m2w2 = _pad2(m2_W2, 32, 32)
    m2b2 = jnp.pad(m2_b2, (0, 2)).reshape(1, 32)
    m2m = _make_m(m2_Wk, m2_bk, 32, 64).astype(jnp.bfloat16)

    eye = jnp.eye(32, dtype=jnp.bfloat16)
    rx = jnp.kron(jnp.ones((1, 32), jnp.bfloat16), eye)
    rh = jnp.kron(eye, jnp.ones((1, 32), jnp.bfloat16))

    xg = _sc_gather(x32, src2)
    (msg1,) = _edge_call(e_pad, xg, m1w1, m1b1, m1w2, m1b2, rx, rh, m1m, 32)
    agg1 = _sc_scatter(msg1, dsts, z32, 32)
    h1 = _node1_call(agg1, x, m1_root, m1_bias.reshape(1, 32))

    h1g = _sc_gather(h1, src2)
    msg2a, msg2b = _edge_call(e_pad, h1g, m2w1, m2b1, m2w2, m2b2, rx, rh, m2m, 64)
    agg2a = _sc_scatter(msg2a, dsts, z32, 32)
    agg2b = _sc_scatter(msg2b, dsts, z32, 32)
    hw = _node2_call(agg2a, agg2b, h1, m2_root, m2_bias.reshape(1, 64), gcn_W)

    agg3 = _sc_gather_scatter(hw, src2, dsts, z32)
    i3 = i.reshape(N // BN, 1, BN)
    return _pool_call(agg3, i3, gcn_b.reshape(1, 32))


# speedup vs baseline: 3.5565x; 1.0158x over previous
import functools

import jax
import jax.numpy as jnp
from jax import lax
from jax.experimental import pallas as pl
from jax.experimental.pallas import tpu as pltpu
from jax.experimental.pallas import tpu_sc as plsc

N = 20000
E = 50000
F_IN = 30
D_E = 16
N_GRAPHS = 512

T_E = 4096
EPAD = 53248
BN = 2000
NPAD = 20480

NC = 2
NS = 16
NW = NC * NS
CHUNK = 128
CH = EPAD // (NW * CHUNK)
PERW = CH * CHUNK
HALF = NPAD // 2
IMG = HALF + 128
SSTRIPE = IMG // NS
DSTRIPE = HALF // NS


def _relu(v):
    return jnp.maximum(v, 0.0)


def _sc_mesh():
    return plsc.VectorSubcoreMesh(core_axis_name="c", subcore_axis_name="s")


def _sc_gather(table, idx2):
    @functools.partial(
        pl.kernel,
        out_type=jax.ShapeDtypeStruct((EPAD, 32), jnp.bfloat16),
        mesh=_sc_mesh(),
        compiler_params=pltpu.CompilerParams(use_tc_tiling_on_sc=False),
        scratch_types=[
            pltpu.VMEM((PERW,), jnp.int32),
            pltpu.VMEM((PERW, 32), jnp.bfloat16),
            pltpu.SemaphoreType.DMA,
        ],
    )
    def k(table_hbm, idx_hbm, out_hbm, idx_v, rows_v, sem):
        wid = lax.axis_index("s") * NC + lax.axis_index("c")
        pltpu.sync_copy(idx_hbm.at[wid], idx_v)
        pltpu.async_copy(table_hbm.at[idx_v], rows_v, sem).wait()
        pltpu.sync_copy(rows_v, out_hbm.at[pl.ds(wid * PERW, PERW)])

    return k(table, idx2)


def _sc_scatter(msg, dsts, zrows, c):
    @functools.partial(
        pl.kernel,
        out_type=jax.ShapeDtypeStruct((NPAD, c), jnp.bfloat16),
        mesh=_sc_mesh(),
        compiler_params=pltpu.CompilerParams(use_tc_tiling_on_sc=False),
        scratch_types=[
            pltpu.VMEM((CH, CHUNK), jnp.int32),
            pltpu.VMEM((PERW, c), jnp.bfloat16),
            pltpu.VMEM_SHARED((IMG, c), jnp.bfloat16),
        ],
    )
    def k(msg_hbm, dst_hbm, z_hbm, out_hbm, dst_v, msg_v, shared):
        cc = lax.axis_index("c")
        s = lax.axis_index("s")
        pltpu.sync_copy(z_hbm.at[pl.ds(s * SSTRIPE, SSTRIPE)],
                        shared.at[pl.ds(s * SSTRIPE, SSTRIPE)])
        plsc.subcore_barrier()
        for half in range(2):
            w = s * 2 + half
            pltpu.sync_copy(dst_hbm.at[cc].at[w], dst_v)
            pltpu.sync_copy(msg_hbm.at[pl.ds(w * PERW, PERW)], msg_v)
            for j in range(CH):
                pltpu.sync_copy(msg_v.at[pl.ds(j * CHUNK, CHUNK)],
                                shared.at[dst_v.at[j]], add=True)
        plsc.subcore_barrier()
        pltpu.sync_copy(shared.at[pl.ds(s * DSTRIPE, DSTRIPE)],
                        out_hbm.at[pl.ds(cc * HALF + s * DSTRIPE, DSTRIPE)])

    return k(msg, dsts, zrows)


def _sc_gather_scatter(hw, src2, dsts, zrows):
    @functools.partial(
        pl.kernel,
        out_type=jax.ShapeDtypeStruct((NPAD, 32), jnp.bfloat16),
        mesh=_sc_mesh(),
        compiler_params=pltpu.CompilerParams(use_tc_tiling_on_sc=False),
        scratch_types=[
            pltpu.VMEM((PERW,), jnp.int32),
            pltpu.VMEM((CH, CHUNK), jnp.int32),
            pltpu.VMEM((PERW, 32), jnp.bfloat16),
            pltpu.VMEM_SHARED((IMG, 32), jnp.bfloat16),
            pltpu.SemaphoreType.DMA,
        ],
    )
    def k(hw_hbm, src_hbm, dst_hbm, z_hbm, out_hbm, src_v, dst_v, rows_v,
          shared, sem):
        cc = lax.axis_index("c")
        s = lax.axis_index("s")
        pltpu.sync_copy(z_hbm.at[pl.ds(s * SSTRIPE, SSTRIPE)],
                        shared.at[pl.ds(s * SSTRIPE, SSTRIPE)])
        plsc.subcore_barrier()
        for half in range(2):
            w = s * 2 + half
            pltpu.sync_copy(src_hbm.at[w], src_v)
            pltpu.sync_copy(dst_hbm.at[cc].at[w], dst_v)
            pltpu.async_copy(hw_hbm.at[src_v], rows_v, sem).wait()
            for j in range(CH):
                pltpu.sync_copy(rows_v.at[pl.ds(j * CHUNK, CHUNK)],
                                shared.at[dst_v.at[j]], add=True)
        plsc.subcore_barrier()
        pltpu.sync_copy(shared.at[pl.ds(s * DSTRIPE, DSTRIPE)],
                        out_hbm.at[pl.ds(cc * HALF + s * DSTRIPE, DSTRIPE)])

    return k(hw, src2, dsts, zrows)


def _edge_body(e_ref, xg_ref, w1_ref, b1_ref, w2_ref, b2_ref, rx_ref, rh_ref,
               m_ref, *o_refs, c):
    h = _relu(jnp.dot(e_ref[...], w1_ref[...],
                      preferred_element_type=jnp.float32) + b1_ref[...])
    h = _relu(jnp.dot(h, w2_ref[...],
                      preferred_element_type=jnp.float32) + b2_ref[...])
    lane = jax.lax.broadcasted_iota(jnp.int32, h.shape, 1)
    h = jnp.where(lane == 30, 1.0, h)
    xt = jnp.dot(xg_ref[...], rx_ref[...],
                 preferred_element_type=jnp.float32)
    hr = jnp.dot(h.astype(jnp.bfloat16), rh_ref[...],
                 preferred_element_type=jnp.float32)
    p = (xt * hr).astype(jnp.bfloat16)
    msg = jnp.dot(p, m_ref[...], preferred_element_type=jnp.float32)
    row = jax.lax.broadcasted_iota(jnp.int32, msg.shape, 0) + pl.program_id(0) * T_E
    msg = jnp.where(row < E, msg, 0.0)
    for g, o_ref in enumerate(o_refs):
        o_ref[...] = msg[:, g * 32:(g + 1) * 32].astype(jnp.bfloat16)


def _edge_call(e_pad, xg, w1p, b1p, w2p, b2p, rx, rh, mp, c):
    grid = EPAD // T_E
    return pl.pallas_call(
        functools.partial(_edge_body, c=c),
        grid=(grid,),
        in_specs=[
            pl.BlockSpec((T_E, D_E), lambda n: (n, 0)),
            pl.BlockSpec((T_E, 32), lambda n: (n, 0)),
            pl.BlockSpec((D_E, 32), lambda n: (0, 0)),
            pl.BlockSpec((1, 32), lambda n: (0, 0)),
            pl.BlockSpec((32, 32), lambda n: (0, 0)),
            pl.BlockSpec((1, 32), lambda n: (0, 0)),
            pl.BlockSpec((32, 32 * 32), lambda n: (0, 0)),
            pl.BlockSpec((32, 32 * 32), lambda n: (0, 0)),
            pl.BlockSpec((32 * 32, c), lambda n: (0, 0)),
        ],
        out_specs=[pl.BlockSpec((T_E, 32), lambda n: (n, 0))] * (c // 32),
        out_shape=[jax.ShapeDtypeStruct((EPAD, 32), jnp.bfloat16)] * (c // 32),
    )(e_pad, xg, w1p, b1p, w2p, b2p, rx, rh, mp)


def _node1_body(agg_ref, x_ref, root_ref, bias_ref, o_ref):
    o_ref[...] = _relu(
        agg_ref[...].astype(jnp.float32)
        + jnp.dot(x_ref[...], root_ref[...], preferred_element_type=jnp.float32)
        + bias_ref[...]).astype(jnp.bfloat16)


def _node1_call(agg, x, root, bias):
    grid = N // BN
    return pl.pallas_call(
        _node1_body,
        grid=(grid,),
        in_specs=[
            pl.BlockSpec((BN, 32), lambda n: (n, 0)),
            pl.BlockSpec((BN, F_IN), lambda n: (n, 0)),
            pl.BlockSpec((F_IN, 32), lambda n: (0, 0)),
            pl.BlockSpec((1, 32), lambda n: (0, 0)),
        ],
        out_specs=pl.BlockSpec((BN, 32), lambda n: (n, 0)),
        out_shape=jax.ShapeDtypeStruct((N, 32), jnp.bfloat16),
    )(agg, x, root, bias)


def _node2_body(agga_ref, aggb_ref, h1_ref, root_ref, bias_ref, gw_ref, o_ref):
    rt = jnp.dot(h1_ref[...].astype(jnp.float32), root_ref[...],
                 preferred_element_type=jnp.float32)
    h2a = _relu(agga_ref[...].astype(jnp.float32) + rt[:, :32]
                + bias_ref[...][:, :32])
    h2b = _relu(aggb_ref[...].astype(jnp.float32) + rt[:, 32:]
                + bias_ref[...][:, 32:])
    o_ref[...] = (
        jnp.dot(h2a, gw_ref[...][:32], preferred_element_type=jnp.float32)
        + jnp.dot(h2b, gw_ref[...][32:], preferred_element_type=jnp.float32)
    ).astype(jnp.bfloat16)


def _node2_call(agga, aggb, h1, root, bias, gw):
    grid = N // BN
    return pl.pallas_call(
        _node2_body,
        grid=(grid,),
        in_specs=[
            pl.BlockSpec((BN, 32), lambda n: (n, 0)),
            pl.BlockSpec((BN, 32), lambda n: (n, 0)),
            pl.BlockSpec((BN, 32), lambda n: (n, 0)),
            pl.BlockSpec((32, 64), lambda n: (0, 0)),
            pl.BlockSpec((1, 64), lambda n: (0, 0)),
            pl.BlockSpec((64, 32), lambda n: (0, 0)),
        ],
        out_specs=pl.BlockSpec((BN, 32), lambda n: (n, 0)),
        out_shape=jax.ShapeDtypeStruct((N, 32), jnp.bfloat16),
    )(agga, aggb, h1, root, bias, gw)


def _pool_body(agg_ref, i_ref, b_ref, o_ref):
    n = pl.program_id(0)
    h3 = _relu(agg_ref[...].astype(jnp.float32) + b_ref[...])
    iv = i_ref[0]
    onehot = jnp.where(
        jax.lax.broadcasted_iota(jnp.int32, (N_GRAPHS, BN), 0) == iv, 1.0, 0.0)
    contrib = jnp.dot(onehot, h3, preferred_element_type=jnp.float32)

    @pl.when(n == 0)
    def _():
        o_ref[...] = jnp.zeros_like(o_ref)

    o_ref[...] += contrib


def _pool_call(agg3, i3, gcn_b):
    grid = N // BN
    return pl.pallas_call(
        _pool_body,
        grid=(grid,),
        in_specs=[
            pl.BlockSpec((BN, 32), lambda n: (n, 0)),
            pl.BlockSpec((1, 1, BN), lambda n: (n, 0, 0)),
            pl.BlockSpec((1, 32), lambda n: (0, 0)),
        ],
        out_specs=pl.BlockSpec((N_GRAPHS, 32), lambda n: (0, 0)),
        out_shape=jax.ShapeDtypeStruct((N_GRAPHS, 32), jnp.float32),
    )(agg3, i3, gcn_b)


def _make_m(wk, bk, f, c):
    wk_aug = jnp.concatenate(
        [wk, bk[None, :], jnp.zeros((1, f * c), jnp.float32)], axis=0)
    a = wk_aug.reshape(32, f, c)
    if f < 32:
        a = jnp.pad(a, ((0, 0), (0, 32 - f), (0, 0)))
    return a.reshape(32 * 32, c)


def _pad2(w, rows, cols):
    return jnp.pad(w, ((0, rows - w.shape[0]), (0, cols - w.shape[1])))


def kernel(x, edge_index, e, i, m1_W1, m1_b1, m1_W2, m1_b2, m1_Wk, m1_bk,
           m1_root, m1_bias, m2_W1, m2_b1, m2_W2, m2_b2, m2_Wk, m2_bk,
           m2_root, m2_bias, gcn_W, gcn_b):
    src = edge_index[0]
    dst = edge_index[1]
    src2 = jnp.concatenate(
        [src, jnp.zeros((EPAD - E,), jnp.int32)]).reshape(NW, PERW)
    dpad = jnp.concatenate([dst, jnp.full((EPAD - E,), NPAD - 1, jnp.int32)])
    d0 = jnp.where(dpad < HALF, dpad, HALF)
    d1r = dpad - HALF
    d1 = jnp.where((d1r >= 0) & (d1r < HALF), d1r, HALF)
    dsts = jnp.stack([d0, d1]).reshape(NC, NW, CH, CHUNK)
    z32 = jnp.zeros((IMG, 32), jnp.bfloat16)

    e_pad = jnp.pad(e, ((0, EPAD - E), (0, 0))).astype(jnp.bfloat16)
    x32 = jnp.pad(x, ((0, 0), (0, 32 - F_IN))).astype(jnp.bfloat16)

    m1w1 = _pad2(m1_W1, D_E, 32).astype(jnp.bfloat16)
    m1b1 = jnp.pad(m1_b1, (0, 2)).reshape(1, 32)
    m1w2 = _pad2(m1_W2, 32, 32)
    m1b2 = jnp.pad(m1_b2, (0, 2)).reshape(1, 32)
    m1m = _make_m(m1_Wk, m1_bk, F_IN, 32).astype(jnp.bfloat16)
    m2w1 = _pad2(m2_W1, D_E, 32).astype(jnp.bfloat16)
    m2b1 = jnp.pad(m2_b1, (0, 2)).reshape(1, 32)
    m2w2 = _pad2(m2_W2, 32, 32)
    m2b2 = jnp.pad(m2_b2, (0, 2)).reshape(1, 32)
    m2m = _make_m(m2_Wk, m2_bk, 32, 64).astype(jnp.bfloat16)

    eye = jnp.eye(32, dtype=jnp.bfloat16)
    rx = jnp.kron(jnp.ones((1, 32), jnp.bfloat16), eye)
    rh = jnp.kron(eye, jnp.ones((1, 32), jnp.bfloat16))

    xg = _sc_gather(x32, src2)
    (msg1,) = _edge_call(e_pad, xg, m1w1, m1b1, m1w2, m1b2, rx, rh, m1m, 32)
    agg1 = _sc_scatter(msg1, dsts, z32, 32)
    h1 = _node1_call(agg1, x, m1_root, m1_bias.reshape(1, 32))

    h1g = _sc_gather(h1, src2)
    msg2a, msg2b = _edge_call(e_pad, h1g, m2w1, m2b1, m2w2, m2b2, rx, rh, m2m, 64)
    agg2a = _sc_scatter(msg2a, dsts, z32, 32)
    agg2b = _sc_scatter(msg2b, dsts, z32, 32)
    hw = _node2_call(agg2a, agg2b, h1, m2_root, m2_bias.reshape(1, 64), gcn_W)

    agg3 = _sc_gather_scatter(hw, src2, dsts, z32)
    i3 = i.reshape(N // BN, 1, BN)
    return _pool_call(agg3, i3, gcn_b.reshape(1, 32))
